# Initial kernel scaffold; baseline (speedup 1.0000x reference)
import jax
import jax.numpy as jnp
from jax.experimental import pallas as pl


def kernel(x, edge_attr, W_node, b_node, W_edge, b_edge, W_enc, b_enc, W_pre, b_pre, W_post, b_post, W_lin, b_lin, bn_g, bn_b, W1, b1, W2, b2, W3, b3, edge_index):
    raise NotImplementedError("write your pallas kernel here")


# speedup vs baseline: 2.2509x; 2.2509x over previous
import functools
import jax
import jax.numpy as jnp
import numpy as np
from jax.experimental import pallas as pl

N_NODES = 10000
TOWERS = 5
F = 40
LAYERS = 2
DEG_HIST = np.zeros(64, dtype=np.float64)
DEG_HIST[32] = float(N_NODES)
AVG_LOG = float((np.log(np.arange(64) + 1.0) * DEG_HIST).sum() / DEG_HIST.sum())


def _edge_mlp_body(zin_ref, w1_ref, b1_ref, w2_ref, b2_ref, w3_ref, b3_ref, out_ref):
    z = zin_ref[...]
    z = jax.nn.relu(z @ w1_ref[...] + b1_ref[...])
    z = jax.nn.relu(z @ w2_ref[...] + b2_ref[...])
    out_ref[...] = z @ w3_ref[...] + b3_ref[...]


def _edge_mlp(zin, W1, b1, W2, b2, W3, b3):
    E = zin.shape[0]
    BLK = 8000
    grid = (E // BLK,)
    return pl.pallas_call(
        _edge_mlp_body,
        grid=grid,
        in_specs=[
            pl.BlockSpec((BLK, zin.shape[1]), lambda i: (i, 0)),
            pl.BlockSpec(W1.shape, lambda i: (0, 0)),
            pl.BlockSpec(b1.shape, lambda i: (0,)),
            pl.BlockSpec(W2.shape, lambda i: (0, 0)),
            pl.BlockSpec(b2.shape, lambda i: (0,)),
            pl.BlockSpec(W3.shape, lambda i: (0, 0)),
            pl.BlockSpec(b3.shape, lambda i: (0,)),
        ],
        out_specs=pl.BlockSpec((BLK, 2), lambda i: (i, 0)),
        out_shape=jax.ShapeDtypeStruct((E, 2), jnp.float32),
    )(zin, W1, b1, W2, b2, W3, b3)


def kernel(x, edge_attr, W_node, b_node, W_edge, b_edge, W_enc, b_enc, W_pre, b_pre, W_post, b_post, W_lin, b_lin, bn_g, bn_b, W1, b1, W2, b2, W3, b3, edge_index):
    src = edge_index[0]
    dst = edge_index[1]
    h = x @ W_node + b_node
    ea = edge_attr @ W_edge + b_edge

    deg = jnp.bincount(dst, length=N_NODES).astype(jnp.float32)
    has = (deg > 0)[:, None]
    degc = jnp.clip(deg, 1.0, None)[:, None]
    amp = jnp.log(degc + 1.0) / AVG_LOG
    att = AVG_LOG / jnp.log(degc + 1.0)

    for i in range(LAYERS):
        Wd = W_pre[i][:, :F, :]
        Ws = W_pre[i][:, F:2*F, :]
        We = W_pre[i][:, 2*F:, :]
        C = jnp.einsum('hf,tfg->thg', W_enc[i], We)
        bD = jnp.einsum('f,tfg->tg', b_enc[i], We) + b_pre[i]
        P = jnp.concatenate([h @ Wd[t] for t in range(TOWERS)], axis=-1)
        Q = jnp.concatenate([h @ Ws[t] for t in range(TOWERS)], axis=-1)
        D = jnp.concatenate([ea @ C[t] + bD[t] for t in range(TOWERS)], axis=-1)

        u = Q[src] + D
        S1 = jax.ops.segment_sum(u, dst, num_segments=N_NODES)
        S2 = jax.ops.segment_sum(u * u, dst, num_segments=N_NODES)
        MN = jax.ops.segment_min(u, dst, num_segments=N_NODES)
        MX = jax.ops.segment_max(u, dst, num_segments=N_NODES)

        mean = jnp.where(has, P + S1 / degc, 0.0)
        mn = jnp.where(has, P + MN, 0.0)
        mx = jnp.where(has, P + MX, 0.0)
        s1d = S1 / degc
        std = jnp.sqrt(jax.nn.relu(S2 / degc - s1d * s1d) + 1e-5)

        outs = []
        for t in range(TOWERS):
            sl = slice(t*F, (t+1)*F)
            agg = jnp.concatenate([mean[:, sl], mn[:, sl], mx[:, sl], std[:, sl]], axis=-1)
            scaled = jnp.concatenate([agg, agg*amp, agg*att], axis=-1)
            outs.append(jnp.concatenate([h, scaled], axis=-1) @ W_post[i][t] + b_post[i][t])
        c = jnp.concatenate(outs, axis=-1) @ W_lin[i] + b_lin[i]
        mu = c.mean(axis=0)
        var = c.var(axis=0)
        cbn = bn_g[i] * (c - mu) / jnp.sqrt(var + 1e-5) + bn_b[i]
        h = (h + jax.nn.relu(cbn)) / 2.0

    hr = jax.nn.relu(h)
    G1 = hr @ W1[:40]
    G2 = hr @ W1[40:80]
    EA1 = ea @ W1[80:]
    zin = G1[src] + G2[dst] + EA1
    zin = jax.nn.relu(zin + b1)
    out = _edge_mlp_tail(zin, W2, b2, W3, b3)
    return out


def _edge_mlp_tail_body(zin_ref, w2_ref, b2_ref, w3_ref, b3_ref, out_ref):
    z = jax.nn.relu(zin_ref[...] @ w2_ref[...] + b2_ref[...])
    out_ref[...] = z @ w3_ref[...] + b3_ref[...]


def _edge_mlp_tail(zin, W2, b2, W3, b3):
    E = zin.shape[0]
    BLK = 8000
    grid = (E // BLK,)
    return pl.pallas_call(
        _edge_mlp_tail_body,
        grid=grid,
        in_specs=[
            pl.BlockSpec((BLK, zin.shape[1]), lambda i: (i, 0)),
            pl.BlockSpec(W2.shape, lambda i: (0, 0)),
            pl.BlockSpec(b2.shape, lambda i: (0,)),
            pl.BlockSpec(W3.shape, lambda i: (0, 0)),
            pl.BlockSpec(b3.shape, lambda i: (0,)),
        ],
        out_specs=pl.BlockSpec((BLK, 2), lambda i: (i, 0)),
        out_shape=jax.ShapeDtypeStruct((E, 2), jnp.float32),
    )(zin, W2, b2, W3, b3)


# speedup vs baseline: 3.3726x; 1.4983x over previous
import functools
import jax
import jax.numpy as jnp
import numpy as np
from jax import lax
from jax.experimental import pallas as pl
from jax.experimental.pallas import tpu as pltpu
from jax.experimental.pallas import tpu_sc as plsc

N_NODES = 10000
N_EDGES = 320000
TOWERS = 5
F = 40
LAYERS = 2
DEG_HIST = np.zeros(64, dtype=np.float64)
DEG_HIST[32] = float(N_NODES)
AVG_LOG = float((np.log(np.arange(64) + 1.0) * DEG_HIST).sum() / DEG_HIST.sum())

NG = 128
GSZ = 79
NNP = NG * GSZ
W = 256
DEGC = 200
EBLK = 64
EP = N_EDGES + EBLK
NB = 144
GW = 128
HIGH = jax.lax.Precision.HIGHEST


_sc_mesh = plsc.VectorSubcoreMesh(core_axis_name="c", subcore_axis_name="s")


@functools.partial(
    pl.kernel,
    mesh=_sc_mesh,
    out_type=[jax.ShapeDtypeStruct((NNP * W,), jnp.float32)] * 4,
    scratch_types=[
        pltpu.VMEM((GSZ * W,), jnp.float32),
        pltpu.VMEM((GSZ * W,), jnp.float32),
        pltpu.VMEM((GSZ * W,), jnp.float32),
        pltpu.VMEM((GSZ * W,), jnp.float32),
        pltpu.VMEM((EBLK,), jnp.int32),
        pltpu.VMEM((EBLK,), jnp.int32),
        pltpu.VMEM((EBLK, W), jnp.float32),
        pltpu.VMEM((EBLK, W), jnp.float32),
        pltpu.VMEM((EBLK + 16,), jnp.float32),
        pltpu.VMEM((NB,), jnp.float32),
        pltpu.SemaphoreType.DMA,
        pltpu.SemaphoreType.DMA,
    ],
)
def _sc_segment(q, d, dsts, srcs, perms, bounds,
                s1o, s2o, mno, mxo,
                a1, a2, a3, a4, srcv, permv, qbuf, dbuf, dstv, bndv,
                sem0, sem1):
    wid = lax.axis_index("c") * 16 + lax.axis_index("s")
    pltpu.sync_copy(bounds, bndv)
    zero = jnp.zeros((16,), jnp.float32)
    big = jnp.full((16,), 1e30, jnp.float32)

    def _extract(vecref, idx):
        return vecref[pl.ds(idx, 16)][0]

    def _bval(idx):
        return _extract(bndv, idx).astype(jnp.int32)

    for gi in range(NG // 32):
        g = wid * (NG // 32) + gi
        gbase = g * GSZ
        e0 = _bval(g)
        e1 = _bval(g + 1)

        def init_body(j, _):
            o = j * 16
            a1[pl.ds(o, 16)] = zero
            a2[pl.ds(o, 16)] = zero
            a3[pl.ds(o, 16)] = big
            a4[pl.ds(o, 16)] = -big
            return 0

        lax.fori_loop(0, GSZ * W // 16, init_body, 0)

        b0 = e0 // EBLK
        b1 = (e1 + EBLK - 1) // EBLK

        def blk_body(b, _):
            eb = b * EBLK
            pltpu.sync_copy(dsts.at[pl.ds(eb, EBLK)], dstv.at[pl.ds(0, EBLK)])
            pltpu.sync_copy(srcs.at[pl.ds(eb, EBLK)], srcv)
            pltpu.sync_copy(perms.at[pl.ds(eb, EBLK)], permv)
            c1 = pltpu.async_copy(q.at[srcv], qbuf, sem0)
            c2 = pltpu.async_copy(d.at[permv], dbuf, sem1)
            c1.wait()
            c2.wait()
            lo = jnp.maximum(e0 - eb, 0)
            hi = jnp.minimum(jnp.int32(EBLK), e1 - eb)

            def edge_body(i, _):
                off = _extract(dstv, i).astype(jnp.int32) - gbase * W
                for c in range(W // 16):
                    u = qbuf[i, pl.ds(c * 16, 16)] + dbuf[i, pl.ds(c * 16, 16)]
                    o = off + c * 16
                    a1[pl.ds(o, 16)] = a1[pl.ds(o, 16)] + u
                    a2[pl.ds(o, 16)] = a2[pl.ds(o, 16)] + u * u
                    a3[pl.ds(o, 16)] = jnp.minimum(a3[pl.ds(o, 16)], u)
                    a4[pl.ds(o, 16)] = jnp.maximum(a4[pl.ds(o, 16)], u)
                return 0

            lax.fori_loop(lo, hi, edge_body, 0)
            return 0

        lax.fori_loop(b0, b1, blk_body, 0)

        obase = gbase * W
        pltpu.sync_copy(a1, s1o.at[pl.ds(obase, GSZ * W)])
        pltpu.sync_copy(a2, s2o.at[pl.ds(obase, GSZ * W)])
        pltpu.sync_copy(a3, mno.at[pl.ds(obase, GSZ * W)])
        pltpu.sync_copy(a4, mxo.at[pl.ds(obase, GSZ * W)])


NBLK_G = N_EDGES // EBLK


@functools.partial(
    pl.kernel,
    mesh=_sc_mesh,
    out_type=jax.ShapeDtypeStruct((N_EDGES * GW,), jnp.float32),
    scratch_types=[
        pltpu.VMEM((EBLK,), jnp.int32),
        pltpu.VMEM((EBLK,), jnp.int32),
        pltpu.VMEM((EBLK, GW), jnp.float32),
        pltpu.VMEM((EBLK, GW), jnp.float32),
        pltpu.VMEM((EBLK * GW,), jnp.float32),
        pltpu.SemaphoreType.DMA,
        pltpu.SemaphoreType.DMA,
    ],
)
def _sc_edge_gather(g1, g2, srcs, dsts, out,
                    srcv, dstv, b1, b2, wbuf, sem0, sem1):
    wid = lax.axis_index("c") * 16 + lax.axis_index("s")
    nper = (NBLK_G + 31) // 32

    def blk_body(k, _):
        b = wid * nper + k

        @pl.when(b < NBLK_G)
        def _():
            eb = b * EBLK
            pltpu.sync_copy(srcs.at[pl.ds(eb, EBLK)], srcv)
            pltpu.sync_copy(dsts.at[pl.ds(eb, EBLK)], dstv)
            c1 = pltpu.async_copy(g1.at[srcv], b1, sem0)
            c2 = pltpu.async_copy(g2.at[dstv], b2, sem1)
            c1.wait()
            c2.wait()

            def add_body(j, _):
                o = j * 16
                r = j // (GW // 16)
                c = (j % (GW // 16)) * 16
                wbuf[pl.ds(o, 16)] = b1[r, pl.ds(c, 16)] + b2[r, pl.ds(c, 16)]
                return 0

            lax.fori_loop(0, EBLK * GW // 16, add_body, 0)
            pltpu.sync_copy(wbuf, out.at[pl.ds(eb * GW, EBLK * GW)])

        return 0

    lax.fori_loop(0, nper, blk_body, 0)


def _pre_node_body(x_ref, wn_ref, bn_ref, wq_ref, h_ref, q_ref):
    h = x_ref[...] @ wn_ref[...] + bn_ref[...]
    h_ref[...] = h
    q_ref[...] = h @ wq_ref[...]


def _pre_node(x, Wn, bn, Wq):
    return pl.pallas_call(
        _pre_node_body,
        out_shape=[
            jax.ShapeDtypeStruct((N_NODES, F), jnp.float32),
            jax.ShapeDtypeStruct((N_NODES, W), jnp.float32),
        ],
    )(x, Wn, bn, Wq)


def _pre_edge_body(ea_ref, w_ref, b_ref, wea_ref, bea_ref, d0, d1, ea1):
    a = ea_ref[...]
    w = w_ref[...]
    b = b_ref[...]
    d0[...] = a @ w[0] + b[0]
    d1[...] = a @ w[1] + b[1]
    ea1[...] = a @ wea_ref[...] + bea_ref[...]


def _pre_edge(edge_attr, Wcat, bcat, Wea, bea):
    BLK = 4000
    grid = (N_EDGES // BLK,)
    return pl.pallas_call(
        _pre_edge_body,
        grid=grid,
        in_specs=[
            pl.BlockSpec((BLK, 16), lambda i: (i, 0)),
            pl.BlockSpec(Wcat.shape, lambda i: (0, 0, 0)),
            pl.BlockSpec(bcat.shape, lambda i: (0, 0)),
            pl.BlockSpec(Wea.shape, lambda i: (0, 0)),
            pl.BlockSpec(bea.shape, lambda i: (0,)),
        ],
        out_specs=[pl.BlockSpec((BLK, W), lambda i: (i, 0))] * 2
        + [pl.BlockSpec((BLK, GW), lambda i: (i, 0))],
        out_shape=[jax.ShapeDtypeStruct((N_EDGES, W), jnp.float32)] * 2
        + [jax.ShapeDtypeStruct((N_EDGES, GW), jnp.float32)],
    )(edge_attr, Wcat, bcat, Wea, bea)


def _node_a_body(h_ref, s1_ref, s2_ref, mn_ref, mx_ref, deg_ref,
                 wd_ref, wpx_ref, wa_ref, wb_ref, wc_ref, bp_ref,
                 wl_ref, bl_ref, c_ref):
    h = h_ref[...]
    deg = deg_ref[...]
    has = deg > 0
    degc = jnp.maximum(deg, 1.0)
    P = h @ wd_ref[...]
    S1 = s1_ref[...]
    S2 = s2_ref[...]
    mean = jnp.where(has, P + S1 / degc, 0.0)
    mn = jnp.where(has, P + mn_ref[...], 0.0)
    mx = jnp.where(has, P + mx_ref[...], 0.0)
    s1d = S1 / degc
    std = jnp.sqrt(jax.nn.relu(S2 / degc - s1d * s1d) + 1e-5)
    lg = jnp.log(degc + 1.0)
    amp = lg / AVG_LOG
    att = AVG_LOG / lg
    parts = []
    for t in range(TOWERS):
        sl = slice(t * F, (t + 1) * F)
        parts.extend([mean[:, sl], mn[:, sl], mx[:, sl], std[:, sl]])
    agg = jnp.concatenate(parts, axis=-1)
    c = (h @ wpx_ref[...] + agg @ wa_ref[...]
         + amp * (agg @ wb_ref[...]) + att * (agg @ wc_ref[...]) + bp_ref[...])
    c_ref[...] = c @ wl_ref[...] + bl_ref[...]


def _node_a(h, S1, S2, MN, MX, deg, Wd, Wpx, WA, WB, WC, bp, Wl, bl):
    BLK = 2000
    grid = (N_NODES // BLK,)
    return pl.pallas_call(
        _node_a_body,
        grid=grid,
        in_specs=[
            pl.BlockSpec((BLK, F), lambda i: (i, 0)),
            pl.BlockSpec((BLK, 200), lambda i: (i, 0)),
            pl.BlockSpec((BLK, 200), lambda i: (i, 0)),
            pl.BlockSpec((BLK, 200), lambda i: (i, 0)),
            pl.BlockSpec((BLK, 200), lambda i: (i, 0)),
            pl.BlockSpec((BLK, 1), lambda i: (i, 0)),
            pl.BlockSpec(Wd.shape, lambda i: (0, 0)),
            pl.BlockSpec(Wpx.shape, lambda i: (0, 0)),
            pl.BlockSpec(WA.shape, lambda i: (0, 0)),
            pl.BlockSpec(WB.shape, lambda i: (0, 0)),
            pl.BlockSpec(WC.shape, lambda i: (0, 0)),
            pl.BlockSpec(bp.shape, lambda i: (0,)),
            pl.BlockSpec(Wl.shape, lambda i: (0, 0)),
            pl.BlockSpec(bl.shape, lambda i: (0,)),
        ],
        out_specs=pl.BlockSpec((BLK, F), lambda i: (i, 0)),
        out_shape=jax.ShapeDtypeStruct((N_NODES, F), jnp.float32),
    )(h, S1, S2, MN, MX, deg, Wd, Wpx, WA, WB, WC, bp, Wl, bl)


def _node_b_body(h_ref, c_ref, g_ref, b_ref, wo0_ref, wo1_ref,
                 hn_ref, o0_ref, o1_ref, *, relu_out):
    c = c_ref[...]
    mu = jnp.mean(c, axis=0, keepdims=True)
    var = jnp.mean((c - mu) ** 2, axis=0, keepdims=True)
    cbn = g_ref[...] * (c - mu) / jnp.sqrt(var + 1e-5) + b_ref[...]
    hn = (h_ref[...] + jax.nn.relu(cbn)) / 2.0
    hn_ref[...] = hn
    src_h = jax.nn.relu(hn) if relu_out else hn
    o0_ref[...] = src_h @ wo0_ref[...]
    o1_ref[...] = src_h @ wo1_ref[...]


def _node_b(h, c, bn_g, bn_b, Wo0, Wo1, relu_out):
    return pl.pallas_call(
        functools.partial(_node_b_body, relu_out=relu_out),
        out_shape=[
            jax.ShapeDtypeStruct((N_NODES, F), jnp.float32),
            jax.ShapeDtypeStruct((N_NODES, Wo0.shape[1]), jnp.float32),
            jax.ShapeDtypeStruct((N_NODES, Wo1.shape[1]), jnp.float32),
        ],
    )(h, c, bn_g, bn_b, Wo0, Wo1)


def _edge_final_body(u_ref, ea1_ref, w2_ref, b2_ref, w3_ref, b3_ref, out_ref):
    z = jax.nn.relu(u_ref[...] + ea1_ref[...])
    z = jax.nn.relu(z @ w2_ref[...] + b2_ref[...])
    out_ref[...] = z @ w3_ref[...] + b3_ref[...]


def _edge_final(U12, EA1, W2p, b2, W3, b3):
    BLK = 8000
    grid = (N_EDGES // BLK,)
    return pl.pallas_call(
        _edge_final_body,
        grid=grid,
        in_specs=[
            pl.BlockSpec((BLK, GW), lambda i: (i, 0)),
            pl.BlockSpec((BLK, GW), lambda i: (i, 0)),
            pl.BlockSpec(W2p.shape, lambda i: (0, 0)),
            pl.BlockSpec(b2.shape, lambda i: (0,)),
            pl.BlockSpec(W3.shape, lambda i: (0, 0)),
            pl.BlockSpec(b3.shape, lambda i: (0,)),
        ],
        out_specs=pl.BlockSpec((BLK, 2), lambda i: (i, 0)),
        out_shape=jax.ShapeDtypeStruct((N_EDGES, 2), jnp.float32),
    )(U12, EA1, W2p, b2, W3, b3)


def _fold_layer(W_enc, b_enc, W_pre, b_pre, W_post, b_post, W_edge, b_edge):
    Wd = jnp.concatenate([W_pre[t][:F] for t in range(TOWERS)], axis=1)
    Ws = jnp.concatenate([W_pre[t][F:2*F] for t in range(TOWERS)], axis=1)
    We = [W_pre[t][2*F:] for t in range(TOWERS)]
    C = jnp.concatenate([jnp.matmul(W_enc, We[t], precision=HIGH)
                         for t in range(TOWERS)], axis=1)
    bD = jnp.concatenate([jnp.matmul(b_enc[None], We[t], precision=HIGH)[0]
                          + b_pre[t] for t in range(TOWERS)])
    Cea = jnp.matmul(W_edge, C, precision=HIGH)
    bDe = jnp.matmul(b_edge[None], C, precision=HIGH)[0] + bD
    Wpx = jnp.concatenate([W_post[t][:F] for t in range(TOWERS)], axis=1)
    WA = jax.scipy.linalg.block_diag(*[W_post[t][F:F+4*F] for t in range(TOWERS)])
    WB = jax.scipy.linalg.block_diag(*[W_post[t][F+4*F:F+8*F] for t in range(TOWERS)])
    WC = jax.scipy.linalg.block_diag(*[W_post[t][F+8*F:] for t in range(TOWERS)])
    bp = jnp.concatenate([b_post[t] for t in range(TOWERS)])
    return Wd, Ws, Cea, bDe, Wpx, WA, WB, WC, bp


def _padw(M, width):
    pad = [(0, 0)] * (M.ndim - 1) + [(0, width - M.shape[-1])]
    return jnp.pad(M, pad)


def kernel(x, edge_attr, W_node, b_node, W_edge, b_edge, W_enc, b_enc, W_pre, b_pre, W_post, b_post, W_lin, b_lin, bn_g, bn_b, W1, b1, W2, b2, W3, b3, edge_index):
    src = edge_index[0]
    dst = edge_index[1]

    eid = jnp.arange(N_EDGES, dtype=jnp.int32)
    dst_s, src_s, perm = lax.sort((dst, src, eid), num_keys=1)
    bounds = jnp.searchsorted(
        dst_s, (jnp.arange(NG + 1, dtype=jnp.int32) * GSZ).astype(jnp.int32)
    ).astype(jnp.int32)
    bounds = jnp.pad(bounds, (0, NB - (NG + 1)), constant_values=N_EDGES)
    bounds_f = bounds.astype(jnp.float32)
    zpad = jnp.zeros((EBLK,), jnp.int32)
    dstw_p = jnp.concatenate([(dst_s * W).astype(jnp.float32),
                              jnp.zeros((EBLK,), jnp.float32)])
    src_sp = jnp.concatenate([src_s, zpad])
    perm_p = jnp.concatenate([perm, zpad])

    folds = [
        _fold_layer(W_enc[i], b_enc[i], W_pre[i], b_pre[i],
                    W_post[i], b_post[i], W_edge, b_edge)
        for i in range(LAYERS)
    ]

    Wd_list, bd_list = [], []
    for i in range(LAYERS):
        Cea, bDe = folds[i][2], folds[i][3]
        bpad = _padw(bDe, W)
        if i == 0:
            bpad = bpad.at[DEGC].set(1.0)
        Wd_list.append(_padw(Cea, W))
        bd_list.append(bpad)
    Wcat = jnp.stack(Wd_list)
    bcat = jnp.stack(bd_list)
    W1c = jnp.matmul(W_edge, W1[80:], precision=HIGH)
    b1c = jnp.matmul(b_edge[None], W1[80:], precision=HIGH)[0] + b1
    Wea = _padw(W1c, GW)
    bea = _padw(b1c, GW)

    D0, D1, EA1 = _pre_edge(edge_attr, Wcat, bcat, Wea, bea)
    h, Q = _pre_node(x, W_node, b_node, _padw(folds[0][1], W))

    deg = None
    for i in range(LAYERS):
        Dc = D0 if i == 0 else D1
        s1f, s2f, mnf, mxf = _sc_segment(Q, Dc, dstw_p, src_sp, perm_p,
                                         bounds_f)
        s1r = s1f.reshape(NNP, W)
        s2r = s2f.reshape(NNP, W)
        mnr = mnf.reshape(NNP, W)
        mxr = mxf.reshape(NNP, W)
        if deg is None:
            deg = s1r[:N_NODES, DEGC:DEGC + 1]
        S1 = s1r[:N_NODES, :200]
        S2 = s2r[:N_NODES, :200]
        MN = mnr[:N_NODES, :200]
        MX = mxr[:N_NODES, :200]

        Wd, Ws, _, _, Wpx, WA, WB, WC, bp = folds[i]
        c = _node_a(h, S1, S2, MN, MX, deg, Wd, Wpx, WA, WB, WC, bp,
                    W_lin[i], b_lin[i])
        if i + 1 < LAYERS:
            Wo0 = _padw(folds[i + 1][1], W)
            Wo1 = jnp.zeros((F, 8), jnp.float32)
            h, Q, _ = _node_b(h, c, bn_g[i], bn_b[i], Wo0, Wo1,
                              relu_out=False)
        else:
            Wo0 = _padw(W1[:40], GW)
            Wo1 = _padw(W1[40:80], GW)
            h, G1, G2 = _node_b(h, c, bn_g[i], bn_b[i], Wo0, Wo1,
                                relu_out=True)

    U12 = _sc_edge_gather(G1, G2, src, dst).reshape(N_EDGES, GW)
    W2p = jnp.pad(W2, ((0, GW - 50), (0, 0)))
    return _edge_final(U12, EA1, W2p, b2, W3, b3)


# speedup vs baseline: 3.9154x; 1.1610x over previous
import functools
import jax
import jax.numpy as jnp
import numpy as np
from jax import lax
from jax.experimental import pallas as pl
from jax.experimental.pallas import tpu as pltpu
from jax.experimental.pallas import tpu_sc as plsc

N_NODES = 10000
N_EDGES = 320000
TOWERS = 5
F = 40
LAYERS = 2
DEG_HIST = np.zeros(64, dtype=np.float64)
DEG_HIST[32] = float(N_NODES)
AVG_LOG = float((np.log(np.arange(64) + 1.0) * DEG_HIST).sum() / DEG_HIST.sum())

NG = 128
GSZ = 79
NNP = NG * GSZ
W = 256
AW = 208
DEGC = 200
EBLK = 64
EP = N_EDGES + EBLK
NB = 144
GW = 128
HIGH = jax.lax.Precision.HIGHEST


def _mm(a, b):
    return jnp.matmul(a, b, precision=HIGH)


_sc_mesh = plsc.VectorSubcoreMesh(core_axis_name="c", subcore_axis_name="s")


@functools.partial(
    pl.kernel,
    mesh=_sc_mesh,
    out_type=[jax.ShapeDtypeStruct((NNP * AW,), jnp.float32)] * 4,
    scratch_types=[
        pltpu.VMEM((GSZ * AW,), jnp.float32),
        pltpu.VMEM((GSZ * AW,), jnp.float32),
        pltpu.VMEM((GSZ * AW,), jnp.float32),
        pltpu.VMEM((GSZ * AW,), jnp.float32),
        pltpu.VMEM((EBLK,), jnp.int32),
        pltpu.VMEM((EBLK,), jnp.int32),
        pltpu.VMEM((EBLK, W), jnp.float32),
        pltpu.VMEM((EBLK, W), jnp.float32),
        pltpu.VMEM((EBLK + 16,), jnp.float32),
        pltpu.VMEM((NB,), jnp.float32),
        pltpu.SemaphoreType.DMA,
        pltpu.SemaphoreType.DMA,
    ],
)
def _sc_segment(q, d, dsts, srcs, perms, bounds,
                s1o, s2o, mno, mxo,
                a1, a2, a3, a4, srcv, permv, qbuf, dbuf, dstv, bndv,
                sem0, sem1):
    wid = lax.axis_index("c") * 16 + lax.axis_index("s")
    pltpu.sync_copy(bounds, bndv)
    zero = jnp.zeros((16,), jnp.float32)
    big = jnp.full((16,), 1e30, jnp.float32)

    def _extract(vecref, idx):
        return vecref[pl.ds(idx, 16)][0]

    def _bval(idx):
        return _extract(bndv, idx).astype(jnp.int32)

    for gi in range(NG // 32):
        g = wid * (NG // 32) + gi
        gbase = g * GSZ
        e0 = _bval(g)
        e1 = _bval(g + 1)

        def init_body(j, _):
            o = j * 16
            a1[pl.ds(o, 16)] = zero
            a2[pl.ds(o, 16)] = zero
            a3[pl.ds(o, 16)] = big
            a4[pl.ds(o, 16)] = -big
            return 0

        lax.fori_loop(0, GSZ * AW // 16, init_body, 0)

        b0 = e0 // EBLK
        b1 = (e1 + EBLK - 1) // EBLK

        def blk_body(b, _):
            eb = b * EBLK
            pltpu.sync_copy(dsts.at[pl.ds(eb, EBLK)], dstv.at[pl.ds(0, EBLK)])
            pltpu.sync_copy(srcs.at[pl.ds(eb, EBLK)], srcv)
            pltpu.sync_copy(perms.at[pl.ds(eb, EBLK)], permv)
            c1 = pltpu.async_copy(q.at[srcv], qbuf, sem0)
            c2 = pltpu.async_copy(d.at[permv], dbuf, sem1)
            c1.wait()
            c2.wait()
            lo = jnp.maximum(e0 - eb, 0)
            hi = jnp.minimum(jnp.int32(EBLK), e1 - eb)

            def edge_body(i, _):
                off = _extract(dstv, i).astype(jnp.int32) - gbase * AW
                for c in range(AW // 16):
                    u = qbuf[i, pl.ds(c * 16, 16)] + dbuf[i, pl.ds(c * 16, 16)]
                    o = off + c * 16
                    plsc.addupdate(a1.at[pl.ds(o, 16)], u)
                    plsc.addupdate(a2.at[pl.ds(o, 16)], u * u)
                    a3[pl.ds(o, 16)] = jnp.minimum(a3[pl.ds(o, 16)], u)
                    a4[pl.ds(o, 16)] = jnp.maximum(a4[pl.ds(o, 16)], u)
                return 0

            lax.fori_loop(lo, hi, edge_body, 0)
            return 0

        lax.fori_loop(b0, b1, blk_body, 0)

        obase = gbase * AW
        pltpu.sync_copy(a1, s1o.at[pl.ds(obase, GSZ * AW)])
        pltpu.sync_copy(a2, s2o.at[pl.ds(obase, GSZ * AW)])
        pltpu.sync_copy(a3, mno.at[pl.ds(obase, GSZ * AW)])
        pltpu.sync_copy(a4, mxo.at[pl.ds(obase, GSZ * AW)])


NBLK_G = N_EDGES // EBLK


@functools.partial(
    pl.kernel,
    mesh=_sc_mesh,
    out_type=jax.ShapeDtypeStruct((N_EDGES * GW,), jnp.float32),
    scratch_types=[
        pltpu.VMEM((EBLK,), jnp.int32),
        pltpu.VMEM((EBLK,), jnp.int32),
        pltpu.VMEM((EBLK, GW), jnp.float32),
        pltpu.VMEM((EBLK, GW), jnp.float32),
        pltpu.VMEM((EBLK * GW,), jnp.float32),
        pltpu.SemaphoreType.DMA,
        pltpu.SemaphoreType.DMA,
    ],
)
def _sc_edge_gather(g1, g2, srcs, dsts, out,
                    srcv, dstv, b1, b2, wbuf, sem0, sem1):
    wid = lax.axis_index("c") * 16 + lax.axis_index("s")
    nper = (NBLK_G + 31) // 32

    def blk_body(k, _):
        b = wid * nper + k

        @pl.when(b < NBLK_G)
        def _():
            eb = b * EBLK
            pltpu.sync_copy(srcs.at[pl.ds(eb, EBLK)], srcv)
            pltpu.sync_copy(dsts.at[pl.ds(eb, EBLK)], dstv)
            c1 = pltpu.async_copy(g1.at[srcv], b1, sem0)
            c2 = pltpu.async_copy(g2.at[dstv], b2, sem1)
            c1.wait()
            c2.wait()

            def add_body(j, _):
                o = j * 16
                r = j // (GW // 16)
                c = (j % (GW // 16)) * 16
                wbuf[pl.ds(o, 16)] = b1[r, pl.ds(c, 16)] + b2[r, pl.ds(c, 16)]
                return 0

            lax.fori_loop(0, EBLK * GW // 16, add_body, 0)
            pltpu.sync_copy(wbuf, out.at[pl.ds(eb * GW, EBLK * GW)])

        return 0

    lax.fori_loop(0, nper, blk_body, 0)


def _pre_node_body(x_ref, wn_ref, bn_ref, wq_ref, h_ref, q_ref):
    h = _mm(x_ref[...], wn_ref[...]) + bn_ref[...]
    h_ref[...] = h
    q_ref[...] = _mm(h, wq_ref[...])


def _pre_node(x, Wn, bn, Wq):
    return pl.pallas_call(
        _pre_node_body,
        out_shape=[
            jax.ShapeDtypeStruct((N_NODES, F), jnp.float32),
            jax.ShapeDtypeStruct((N_NODES, W), jnp.float32),
        ],
    )(x, Wn, bn, Wq)


def _pre_edge_body(ea_ref, w_ref, b_ref, wea_ref, bea_ref, d0, d1, ea1):
    a = ea_ref[...]
    w = w_ref[...]
    b = b_ref[...]
    d0[...] = _mm(a, w[0]) + b[0]
    d1[...] = _mm(a, w[1]) + b[1]
    ea1[...] = _mm(a, wea_ref[...]) + bea_ref[...]


def _pre_edge(edge_attr, Wcat, bcat, Wea, bea):
    BLK = 4000
    grid = (N_EDGES // BLK,)
    return pl.pallas_call(
        _pre_edge_body,
        grid=grid,
        in_specs=[
            pl.BlockSpec((BLK, 16), lambda i: (i, 0)),
            pl.BlockSpec(Wcat.shape, lambda i: (0, 0, 0)),
            pl.BlockSpec(bcat.shape, lambda i: (0, 0)),
            pl.BlockSpec(Wea.shape, lambda i: (0, 0)),
            pl.BlockSpec(bea.shape, lambda i: (0,)),
        ],
        out_specs=[pl.BlockSpec((BLK, W), lambda i: (i, 0))] * 2
        + [pl.BlockSpec((BLK, GW), lambda i: (i, 0))],
        out_shape=[jax.ShapeDtypeStruct((N_EDGES, W), jnp.float32)] * 2
        + [jax.ShapeDtypeStruct((N_EDGES, GW), jnp.float32)],
    )(edge_attr, Wcat, bcat, Wea, bea)


def _node_a_body(h_ref, s1_ref, s2_ref, mn_ref, mx_ref, deg_ref,
                 wd_ref, wpx_ref, wa_ref, wb_ref, wc_ref, bp_ref,
                 wl_ref, bl_ref, c_ref):
    h = h_ref[...]
    deg = deg_ref[...]
    has = deg > 0
    degc = jnp.maximum(deg, 1.0)
    P = _mm(h, wd_ref[...])
    S1 = s1_ref[...]
    S2 = s2_ref[...]
    mean = jnp.where(has, P + S1 / degc, 0.0)
    mn = jnp.where(has, P + mn_ref[...], 0.0)
    mx = jnp.where(has, P + mx_ref[...], 0.0)
    s1d = S1 / degc
    std = jnp.sqrt(jax.nn.relu(S2 / degc - s1d * s1d) + 1e-5)
    lg = jnp.log(degc + 1.0)
    amp = lg / AVG_LOG
    att = AVG_LOG / lg
    parts = []
    for t in range(TOWERS):
        sl = slice(t * F, (t + 1) * F)
        parts.extend([mean[:, sl], mn[:, sl], mx[:, sl], std[:, sl]])
    agg = jnp.concatenate(parts, axis=-1)
    c = (_mm(h, wpx_ref[...]) + _mm(agg, wa_ref[...])
         + amp * _mm(agg, wb_ref[...]) + att * _mm(agg, wc_ref[...]) + bp_ref[...])
    c_ref[...] = _mm(c, wl_ref[...]) + bl_ref[...]


def _node_a(h, S1, S2, MN, MX, deg, Wd, Wpx, WA, WB, WC, bp, Wl, bl):
    BLK = 1000
    grid = (N_NODES // BLK,)
    return pl.pallas_call(
        _node_a_body,
        grid=grid,
        in_specs=[
            pl.BlockSpec((BLK, F), lambda i: (i, 0)),
            pl.BlockSpec((BLK, 200), lambda i: (i, 0)),
            pl.BlockSpec((BLK, 200), lambda i: (i, 0)),
            pl.BlockSpec((BLK, 200), lambda i: (i, 0)),
            pl.BlockSpec((BLK, 200), lambda i: (i, 0)),
            pl.BlockSpec((BLK, 1), lambda i: (i, 0)),
            pl.BlockSpec(Wd.shape, lambda i: (0, 0)),
            pl.BlockSpec(Wpx.shape, lambda i: (0, 0)),
            pl.BlockSpec(WA.shape, lambda i: (0, 0)),
            pl.BlockSpec(WB.shape, lambda i: (0, 0)),
            pl.BlockSpec(WC.shape, lambda i: (0, 0)),
            pl.BlockSpec(bp.shape, lambda i: (0,)),
            pl.BlockSpec(Wl.shape, lambda i: (0, 0)),
            pl.BlockSpec(bl.shape, lambda i: (0,)),
        ],
        out_specs=pl.BlockSpec((BLK, F), lambda i: (i, 0)),
        out_shape=jax.ShapeDtypeStruct((N_NODES, F), jnp.float32),
    )(h, S1, S2, MN, MX, deg, Wd, Wpx, WA, WB, WC, bp, Wl, bl)


def _node_b_body(h_ref, c_ref, g_ref, b_ref, wo0_ref, wo1_ref,
                 hn_ref, o0_ref, o1_ref, *, relu_out):
    c = c_ref[...]
    mu = jnp.mean(c, axis=0, keepdims=True)
    var = jnp.mean((c - mu) ** 2, axis=0, keepdims=True)
    cbn = g_ref[...] * (c - mu) / jnp.sqrt(var + 1e-5) + b_ref[...]
    hn = (h_ref[...] + jax.nn.relu(cbn)) / 2.0
    hn_ref[...] = hn
    src_h = jax.nn.relu(hn) if relu_out else hn
    o0_ref[...] = _mm(src_h, wo0_ref[...])
    o1_ref[...] = _mm(src_h, wo1_ref[...])


def _node_b(h, c, bn_g, bn_b, Wo0, Wo1, relu_out):
    return pl.pallas_call(
        functools.partial(_node_b_body, relu_out=relu_out),
        out_shape=[
            jax.ShapeDtypeStruct((N_NODES, F), jnp.float32),
            jax.ShapeDtypeStruct((N_NODES, Wo0.shape[1]), jnp.float32),
            jax.ShapeDtypeStruct((N_NODES, Wo1.shape[1]), jnp.float32),
        ],
    )(h, c, bn_g, bn_b, Wo0, Wo1)


def _edge_final_body(u_ref, ea1_ref, w2_ref, b2_ref, w3_ref, b3_ref, out_ref):
    z = jax.nn.relu(u_ref[...] + ea1_ref[...])
    z = jax.nn.relu(_mm(z, w2_ref[...]) + b2_ref[...])
    out_ref[...] = _mm(z, w3_ref[...]) + b3_ref[...]


def _edge_final(U12, EA1, W2p, b2, W3, b3):
    BLK = 8000
    grid = (N_EDGES // BLK,)
    return pl.pallas_call(
        _edge_final_body,
        grid=grid,
        in_specs=[
            pl.BlockSpec((BLK, GW), lambda i: (i, 0)),
            pl.BlockSpec((BLK, GW), lambda i: (i, 0)),
            pl.BlockSpec(W2p.shape, lambda i: (0, 0)),
            pl.BlockSpec(b2.shape, lambda i: (0,)),
            pl.BlockSpec(W3.shape, lambda i: (0, 0)),
            pl.BlockSpec(b3.shape, lambda i: (0,)),
        ],
        out_specs=pl.BlockSpec((BLK, 2), lambda i: (i, 0)),
        out_shape=jax.ShapeDtypeStruct((N_EDGES, 2), jnp.float32),
    )(U12, EA1, W2p, b2, W3, b3)


def _fold_layer(W_enc, b_enc, W_pre, b_pre, W_post, b_post, W_edge, b_edge):
    Wd = jnp.concatenate([W_pre[t][:F] for t in range(TOWERS)], axis=1)
    Ws = jnp.concatenate([W_pre[t][F:2*F] for t in range(TOWERS)], axis=1)
    We = [W_pre[t][2*F:] for t in range(TOWERS)]
    C = jnp.concatenate([jnp.matmul(W_enc, We[t], precision=HIGH)
                         for t in range(TOWERS)], axis=1)
    bD = jnp.concatenate([jnp.matmul(b_enc[None], We[t], precision=HIGH)[0]
                          + b_pre[t] for t in range(TOWERS)])
    Cea = jnp.matmul(W_edge, C, precision=HIGH)
    bDe = jnp.matmul(b_edge[None], C, precision=HIGH)[0] + bD
    Wpx = jnp.concatenate([W_post[t][:F] for t in range(TOWERS)], axis=1)
    WA = jax.scipy.linalg.block_diag(*[W_post[t][F:F+4*F] for t in range(TOWERS)])
    WB = jax.scipy.linalg.block_diag(*[W_post[t][F+4*F:F+8*F] for t in range(TOWERS)])
    WC = jax.scipy.linalg.block_diag(*[W_post[t][F+8*F:] for t in range(TOWERS)])
    bp = jnp.concatenate([b_post[t] for t in range(TOWERS)])
    return Wd, Ws, Cea, bDe, Wpx, WA, WB, WC, bp


def _padw(M, width):
    pad = [(0, 0)] * (M.ndim - 1) + [(0, width - M.shape[-1])]
    return jnp.pad(M, pad)


def kernel(x, edge_attr, W_node, b_node, W_edge, b_edge, W_enc, b_enc, W_pre, b_pre, W_post, b_post, W_lin, b_lin, bn_g, bn_b, W1, b1, W2, b2, W3, b3, edge_index):
    src = edge_index[0]
    dst = edge_index[1]

    eid = jnp.arange(N_EDGES, dtype=jnp.int32)
    dst_s, src_s, perm = lax.sort((dst, src, eid), num_keys=1)
    bounds = jnp.searchsorted(
        dst_s, (jnp.arange(NG + 1, dtype=jnp.int32) * GSZ).astype(jnp.int32)
    ).astype(jnp.int32)
    bounds = jnp.pad(bounds, (0, NB - (NG + 1)), constant_values=N_EDGES)
    bounds_f = bounds.astype(jnp.float32)
    zpad = jnp.zeros((EBLK,), jnp.int32)
    dstw_p = jnp.concatenate([(dst_s * AW).astype(jnp.float32),
                              jnp.zeros((EBLK,), jnp.float32)])
    src_sp = jnp.concatenate([src_s, zpad])
    perm_p = jnp.concatenate([perm, zpad])

    folds = [
        _fold_layer(W_enc[i], b_enc[i], W_pre[i], b_pre[i],
                    W_post[i], b_post[i], W_edge, b_edge)
        for i in range(LAYERS)
    ]

    Wd_list, bd_list = [], []
    for i in range(LAYERS):
        Cea, bDe = folds[i][2], folds[i][3]
        bpad = _padw(bDe, W)
        if i == 0:
            bpad = bpad.at[DEGC].set(1.0)
        Wd_list.append(_padw(Cea, W))
        bd_list.append(bpad)
    Wcat = jnp.stack(Wd_list)
    bcat = jnp.stack(bd_list)
    W1c = jnp.matmul(W_edge, W1[80:], precision=HIGH)
    b1c = jnp.matmul(b_edge[None], W1[80:], precision=HIGH)[0] + b1
    Wea = _padw(W1c, GW)
    bea = _padw(b1c, GW)

    D0, D1, EA1 = _pre_edge(edge_attr, Wcat, bcat, Wea, bea)
    h, Q = _pre_node(x, W_node, b_node, _padw(folds[0][1], W))

    deg = None
    for i in range(LAYERS):
        Dc = D0 if i == 0 else D1
        s1f, s2f, mnf, mxf = _sc_segment(Q, Dc, dstw_p, src_sp, perm_p,
                                         bounds_f)
        s1r = s1f.reshape(NNP, AW)
        s2r = s2f.reshape(NNP, AW)
        mnr = mnf.reshape(NNP, AW)
        mxr = mxf.reshape(NNP, AW)
        if deg is None:
            deg = s1r[:N_NODES, DEGC:DEGC + 1]
        S1 = s1r[:N_NODES, :200]
        S2 = s2r[:N_NODES, :200]
        MN = mnr[:N_NODES, :200]
        MX = mxr[:N_NODES, :200]

        Wd, Ws, _, _, Wpx, WA, WB, WC, bp = folds[i]
        c = _node_a(h, S1, S2, MN, MX, deg, Wd, Wpx, WA, WB, WC, bp,
                    W_lin[i], b_lin[i])
        if i + 1 < LAYERS:
            Wo0 = _padw(folds[i + 1][1], W)
            Wo1 = jnp.zeros((F, 8), jnp.float32)
            h, Q, _ = _node_b(h, c, bn_g[i], bn_b[i], Wo0, Wo1,
                              relu_out=False)
        else:
            Wo0 = _padw(W1[:40], GW)
            Wo1 = _padw(W1[40:80], GW)
            h, G1, G2 = _node_b(h, c, bn_g[i], bn_b[i], Wo0, Wo1,
                                relu_out=True)

    U12 = _sc_edge_gather(G1, G2, src, dst).reshape(N_EDGES, GW)
    W2p = jnp.pad(W2, ((0, GW - 50), (0, 0)))
    return _edge_final(U12, EA1, W2p, b2, W3, b3)


# speedup vs baseline: 4.2418x; 1.0833x over previous
import functools
import jax
import jax.numpy as jnp
import numpy as np
from jax import lax
from jax.experimental import pallas as pl
from jax.experimental.pallas import tpu as pltpu
from jax.experimental.pallas import tpu_sc as plsc

N_NODES = 10000
N_EDGES = 320000
TOWERS = 5
F = 40
LAYERS = 2
DEG_HIST = np.zeros(64, dtype=np.float64)
DEG_HIST[32] = float(N_NODES)
AVG_LOG = float((np.log(np.arange(64) + 1.0) * DEG_HIST).sum() / DEG_HIST.sum())

NG = 160
GSZ = 64
NNP = NG * GSZ
W = 256
AW = 208
DEGC = 200
EBLK = 64
EP = N_EDGES + EBLK
NB = 192
GW = 128
HIGH = jax.lax.Precision.HIGHEST


def _mm(a, b):
    return jnp.matmul(a, b, precision=HIGH)


_sc_mesh = plsc.VectorSubcoreMesh(core_axis_name="c", subcore_axis_name="s")


@functools.partial(
    pl.kernel,
    mesh=_sc_mesh,
    out_type=[jax.ShapeDtypeStruct((NNP * AW,), jnp.float32)] * 4,
    scratch_types=[
        pltpu.VMEM((GSZ * AW,), jnp.float32),
        pltpu.VMEM((GSZ * AW,), jnp.float32),
        pltpu.VMEM((GSZ * AW,), jnp.float32),
        pltpu.VMEM((GSZ * AW,), jnp.float32),
        pltpu.VMEM((EBLK,), jnp.int32),
        pltpu.VMEM((EBLK,), jnp.int32),
        pltpu.VMEM((EBLK, W), jnp.float32),
        pltpu.VMEM((EBLK, W), jnp.float32),
        pltpu.VMEM((EBLK + 16,), jnp.float32),
        pltpu.VMEM((EBLK,), jnp.int32),
        pltpu.VMEM((EBLK,), jnp.int32),
        pltpu.VMEM((EBLK, W), jnp.float32),
        pltpu.VMEM((EBLK, W), jnp.float32),
        pltpu.VMEM((EBLK + 16,), jnp.float32),
        pltpu.VMEM((NB,), jnp.float32),
        pltpu.SemaphoreType.DMA,
        pltpu.SemaphoreType.DMA,
        pltpu.SemaphoreType.DMA,
        pltpu.SemaphoreType.DMA,
    ],
)
def _sc_segment(q, d, dsts, srcs, perms, bounds,
                s1o, s2o, mno, mxo,
                a1, a2, a3, a4,
                srcvA, permvA, qbufA, dbufA, dstvA,
                srcvB, permvB, qbufB, dbufB, dstvB,
                bndv, semqA, semdA, semqB, semdB):
    wid = lax.axis_index("c") * 16 + lax.axis_index("s")
    pltpu.sync_copy(bounds, bndv)
    zero = jnp.zeros((16,), jnp.float32)
    big = jnp.full((16,), 1e30, jnp.float32)

    bufA = (srcvA, permvA, qbufA, dbufA, dstvA, semqA, semdA)
    bufB = (srcvB, permvB, qbufB, dbufB, dstvB, semqB, semdB)

    def _extract(vecref, idx):
        return vecref[pl.ds(idx, 16)][0]

    def _bval(idx):
        return _extract(bndv, idx).astype(jnp.int32)

    def _fire(buf, b):
        srcv, permv, qbuf, dbuf, dstv, semq, semd = buf
        eb = b * EBLK
        pltpu.sync_copy(dsts.at[pl.ds(eb, EBLK)], dstv.at[pl.ds(0, EBLK)])
        pltpu.sync_copy(srcs.at[pl.ds(eb, EBLK)], srcv)
        pltpu.sync_copy(perms.at[pl.ds(eb, EBLK)], permv)
        pltpu.async_copy(q.at[srcv], qbuf, semq)
        pltpu.async_copy(d.at[permv], dbuf, semd)

    def _wait(buf):
        srcv, permv, qbuf, dbuf, dstv, semq, semd = buf
        pltpu.make_async_copy(q.at[srcv], qbuf, semq).wait()
        pltpu.make_async_copy(d.at[permv], dbuf, semd).wait()

    for gi in range(NG // 32):
        g = wid * (NG // 32) + gi
        gbase = g * GSZ
        e0 = _bval(g)
        e1 = _bval(g + 1)

        def init_body(j, _):
            o = j * 16
            a1[pl.ds(o, 16)] = zero
            a2[pl.ds(o, 16)] = zero
            a3[pl.ds(o, 16)] = big
            a4[pl.ds(o, 16)] = -big
            return 0

        lax.fori_loop(0, GSZ * AW // 16, init_body, 0)

        b0 = e0 // EBLK
        b1 = (e1 + EBLK - 1) // EBLK

        def _process(buf, b):
            srcv, permv, qbuf, dbuf, dstv, semq, semd = buf
            eb = b * EBLK
            lo = jnp.maximum(e0 - eb, 0)
            hi = jnp.minimum(jnp.int32(EBLK), e1 - eb)

            def edge_body(i, _):
                off = _extract(dstv, i).astype(jnp.int32) - gbase * AW
                for c in range(AW // 16):
                    u = qbuf[i, pl.ds(c * 16, 16)] + dbuf[i, pl.ds(c * 16, 16)]
                    o = off + c * 16
                    plsc.addupdate(a1.at[pl.ds(o, 16)], u)
                    plsc.addupdate(a2.at[pl.ds(o, 16)], u * u)
                    a3[pl.ds(o, 16)] = jnp.minimum(a3[pl.ds(o, 16)], u)
                    a4[pl.ds(o, 16)] = jnp.maximum(a4[pl.ds(o, 16)], u)
                return 0

            lax.fori_loop(lo, hi, edge_body, 0)

        _fire(bufA, b0)

        def pair_body(t, _):
            b = b0 + 2 * t
            _fire(bufB, b + 1)
            _wait(bufA)
            _process(bufA, b)
            _fire(bufA, b + 2)
            _wait(bufB)
            _process(bufB, b + 1)
            return 0

        npairs = (b1 - b0 + 1) // 2
        lax.fori_loop(0, npairs, pair_body, 0)
        _wait(bufA)

        obase = gbase * AW
        pltpu.sync_copy(a1, s1o.at[pl.ds(obase, GSZ * AW)])
        pltpu.sync_copy(a2, s2o.at[pl.ds(obase, GSZ * AW)])
        pltpu.sync_copy(a3, mno.at[pl.ds(obase, GSZ * AW)])
        pltpu.sync_copy(a4, mxo.at[pl.ds(obase, GSZ * AW)])


NBLK_G = N_EDGES // EBLK


@functools.partial(
    pl.kernel,
    mesh=_sc_mesh,
    out_type=jax.ShapeDtypeStruct((N_EDGES * GW,), jnp.float32),
    scratch_types=[
        pltpu.VMEM((EBLK,), jnp.int32),
        pltpu.VMEM((EBLK,), jnp.int32),
        pltpu.VMEM((EBLK, GW), jnp.float32),
        pltpu.VMEM((EBLK, GW), jnp.float32),
        pltpu.VMEM((EBLK * GW,), jnp.float32),
        pltpu.SemaphoreType.DMA,
        pltpu.SemaphoreType.DMA,
    ],
)
def _sc_edge_gather(g1, g2, srcs, dsts, out,
                    srcv, dstv, b1, b2, wbuf, sem0, sem1):
    wid = lax.axis_index("c") * 16 + lax.axis_index("s")
    nper = (NBLK_G + 31) // 32

    def blk_body(k, _):
        b = wid * nper + k

        @pl.when(b < NBLK_G)
        def _():
            eb = b * EBLK
            pltpu.sync_copy(srcs.at[pl.ds(eb, EBLK)], srcv)
            pltpu.sync_copy(dsts.at[pl.ds(eb, EBLK)], dstv)
            c1 = pltpu.async_copy(g1.at[srcv], b1, sem0)
            c2 = pltpu.async_copy(g2.at[dstv], b2, sem1)
            c1.wait()
            c2.wait()

            def add_body(j, _):
                o = j * 16
                r = j // (GW // 16)
                c = (j % (GW // 16)) * 16
                wbuf[pl.ds(o, 16)] = b1[r, pl.ds(c, 16)] + b2[r, pl.ds(c, 16)]
                return 0

            lax.fori_loop(0, EBLK * GW // 16, add_body, 0)
            pltpu.sync_copy(wbuf, out.at[pl.ds(eb * GW, EBLK * GW)])

        return 0

    lax.fori_loop(0, nper, blk_body, 0)


def _pre_node_body(x_ref, wn_ref, bn_ref, wq_ref, h_ref, q_ref):
    h = _mm(x_ref[...], wn_ref[...]) + bn_ref[...]
    h_ref[...] = h
    q_ref[...] = _mm(h, wq_ref[...])


def _pre_node(x, Wn, bn, Wq):
    return pl.pallas_call(
        _pre_node_body,
        out_shape=[
            jax.ShapeDtypeStruct((N_NODES, F), jnp.float32),
            jax.ShapeDtypeStruct((N_NODES, W), jnp.float32),
        ],
    )(x, Wn, bn, Wq)


def _pre_edge_body(ea_ref, w_ref, b_ref, wea_ref, bea_ref, d0, d1, ea1):
    a = ea_ref[...]
    w = w_ref[...]
    b = b_ref[...]
    d0[...] = _mm(a, w[0]) + b[0]
    d1[...] = _mm(a, w[1]) + b[1]
    ea1[...] = _mm(a, wea_ref[...]) + bea_ref[...]


def _pre_edge(edge_attr, Wcat, bcat, Wea, bea):
    BLK = 4000
    grid = (N_EDGES // BLK,)
    return pl.pallas_call(
        _pre_edge_body,
        grid=grid,
        in_specs=[
            pl.BlockSpec((BLK, 16), lambda i: (i, 0)),
            pl.BlockSpec(Wcat.shape, lambda i: (0, 0, 0)),
            pl.BlockSpec(bcat.shape, lambda i: (0, 0)),
            pl.BlockSpec(Wea.shape, lambda i: (0, 0)),
            pl.BlockSpec(bea.shape, lambda i: (0,)),
        ],
        out_specs=[pl.BlockSpec((BLK, W), lambda i: (i, 0))] * 2
        + [pl.BlockSpec((BLK, GW), lambda i: (i, 0))],
        out_shape=[jax.ShapeDtypeStruct((N_EDGES, W), jnp.float32)] * 2
        + [jax.ShapeDtypeStruct((N_EDGES, GW), jnp.float32)],
    )(edge_attr, Wcat, bcat, Wea, bea)


def _node_a_body(h_ref, s1_ref, s2_ref, mn_ref, mx_ref, deg_ref,
                 wd_ref, wpx_ref, wa_ref, wb_ref, wc_ref, bp_ref,
                 wl_ref, bl_ref, c_ref):
    h = h_ref[...]
    deg = deg_ref[...]
    has = deg > 0
    degc = jnp.maximum(deg, 1.0)
    P = _mm(h, wd_ref[...])
    S1 = s1_ref[...]
    S2 = s2_ref[...]
    mean = jnp.where(has, P + S1 / degc, 0.0)
    mn = jnp.where(has, P + mn_ref[...], 0.0)
    mx = jnp.where(has, P + mx_ref[...], 0.0)
    s1d = S1 / degc
    std = jnp.sqrt(jax.nn.relu(S2 / degc - s1d * s1d) + 1e-5)
    lg = jnp.log(degc + 1.0)
    amp = lg / AVG_LOG
    att = AVG_LOG / lg
    parts = []
    for t in range(TOWERS):
        sl = slice(t * F, (t + 1) * F)
        parts.extend([mean[:, sl], mn[:, sl], mx[:, sl], std[:, sl]])
    agg = jnp.concatenate(parts, axis=-1)
    c = (_mm(h, wpx_ref[...]) + _mm(agg, wa_ref[...])
         + amp * _mm(agg, wb_ref[...]) + att * _mm(agg, wc_ref[...]) + bp_ref[...])
    c_ref[...] = _mm(c, wl_ref[...]) + bl_ref[...]


def _node_a(h, S1, S2, MN, MX, deg, Wd, Wpx, WA, WB, WC, bp, Wl, bl):
    BLK = 1000
    grid = (N_NODES // BLK,)
    return pl.pallas_call(
        _node_a_body,
        grid=grid,
        in_specs=[
            pl.BlockSpec((BLK, F), lambda i: (i, 0)),
            pl.BlockSpec((BLK, 200), lambda i: (i, 0)),
            pl.BlockSpec((BLK, 200), lambda i: (i, 0)),
            pl.BlockSpec((BLK, 200), lambda i: (i, 0)),
            pl.BlockSpec((BLK, 200), lambda i: (i, 0)),
            pl.BlockSpec((BLK, 1), lambda i: (i, 0)),
            pl.BlockSpec(Wd.shape, lambda i: (0, 0)),
            pl.BlockSpec(Wpx.shape, lambda i: (0, 0)),
            pl.BlockSpec(WA.shape, lambda i: (0, 0)),
            pl.BlockSpec(WB.shape, lambda i: (0, 0)),
            pl.BlockSpec(WC.shape, lambda i: (0, 0)),
            pl.BlockSpec(bp.shape, lambda i: (0,)),
            pl.BlockSpec(Wl.shape, lambda i: (0, 0)),
            pl.BlockSpec(bl.shape, lambda i: (0,)),
        ],
        out_specs=pl.BlockSpec((BLK, F), lambda i: (i, 0)),
        out_shape=jax.ShapeDtypeStruct((N_NODES, F), jnp.float32),
    )(h, S1, S2, MN, MX, deg, Wd, Wpx, WA, WB, WC, bp, Wl, bl)


def _node_b_body(h_ref, c_ref, g_ref, b_ref, wo0_ref, wo1_ref,
                 hn_ref, o0_ref, o1_ref, *, relu_out):
    c = c_ref[...]
    mu = jnp.mean(c, axis=0, keepdims=True)
    var = jnp.mean((c - mu) ** 2, axis=0, keepdims=True)
    cbn = g_ref[...] * (c - mu) / jnp.sqrt(var + 1e-5) + b_ref[...]
    hn = (h_ref[...] + jax.nn.relu(cbn)) / 2.0
    hn_ref[...] = hn
    src_h = jax.nn.relu(hn) if relu_out else hn
    o0_ref[...] = _mm(src_h, wo0_ref[...])
    o1_ref[...] = _mm(src_h, wo1_ref[...])


def _node_b(h, c, bn_g, bn_b, Wo0, Wo1, relu_out):
    return pl.pallas_call(
        functools.partial(_node_b_body, relu_out=relu_out),
        out_shape=[
            jax.ShapeDtypeStruct((N_NODES, F), jnp.float32),
            jax.ShapeDtypeStruct((N_NODES, Wo0.shape[1]), jnp.float32),
            jax.ShapeDtypeStruct((N_NODES, Wo1.shape[1]), jnp.float32),
        ],
    )(h, c, bn_g, bn_b, Wo0, Wo1)


def _edge_final_body(u_ref, ea1_ref, w2_ref, b2_ref, w3_ref, b3_ref, out_ref):
    z = jax.nn.relu(u_ref[...] + ea1_ref[...])
    z = jax.nn.relu(_mm(z, w2_ref[...]) + b2_ref[...])
    out_ref[...] = _mm(z, w3_ref[...]) + b3_ref[...]


def _edge_final(U12, EA1, W2p, b2, W3, b3):
    BLK = 8000
    grid = (N_EDGES // BLK,)
    return pl.pallas_call(
        _edge_final_body,
        grid=grid,
        in_specs=[
            pl.BlockSpec((BLK, GW), lambda i: (i, 0)),
            pl.BlockSpec((BLK, GW), lambda i: (i, 0)),
            pl.BlockSpec(W2p.shape, lambda i: (0, 0)),
            pl.BlockSpec(b2.shape, lambda i: (0,)),
            pl.BlockSpec(W3.shape, lambda i: (0, 0)),
            pl.BlockSpec(b3.shape, lambda i: (0,)),
        ],
        out_specs=pl.BlockSpec((BLK, 2), lambda i: (i, 0)),
        out_shape=jax.ShapeDtypeStruct((N_EDGES, 2), jnp.float32),
    )(U12, EA1, W2p, b2, W3, b3)


def _fold_layer(W_enc, b_enc, W_pre, b_pre, W_post, b_post, W_edge, b_edge):
    Wd = jnp.concatenate([W_pre[t][:F] for t in range(TOWERS)], axis=1)
    Ws = jnp.concatenate([W_pre[t][F:2*F] for t in range(TOWERS)], axis=1)
    We = [W_pre[t][2*F:] for t in range(TOWERS)]
    C = jnp.concatenate([jnp.matmul(W_enc, We[t], precision=HIGH)
                         for t in range(TOWERS)], axis=1)
    bD = jnp.concatenate([jnp.matmul(b_enc[None], We[t], precision=HIGH)[0]
                          + b_pre[t] for t in range(TOWERS)])
    Cea = jnp.matmul(W_edge, C, precision=HIGH)
    bDe = jnp.matmul(b_edge[None], C, precision=HIGH)[0] + bD
    Wpx = jnp.concatenate([W_post[t][:F] for t in range(TOWERS)], axis=1)
    WA = jax.scipy.linalg.block_diag(*[W_post[t][F:F+4*F] for t in range(TOWERS)])
    WB = jax.scipy.linalg.block_diag(*[W_post[t][F+4*F:F+8*F] for t in range(TOWERS)])
    WC = jax.scipy.linalg.block_diag(*[W_post[t][F+8*F:] for t in range(TOWERS)])
    bp = jnp.concatenate([b_post[t] for t in range(TOWERS)])
    return Wd, Ws, Cea, bDe, Wpx, WA, WB, WC, bp


def _padw(M, width):
    pad = [(0, 0)] * (M.ndim - 1) + [(0, width - M.shape[-1])]
    return jnp.pad(M, pad)


def kernel(x, edge_attr, W_node, b_node, W_edge, b_edge, W_enc, b_enc, W_pre, b_pre, W_post, b_post, W_lin, b_lin, bn_g, bn_b, W1, b1, W2, b2, W3, b3, edge_index):
    src = edge_index[0]
    dst = edge_index[1]

    eid = jnp.arange(N_EDGES, dtype=jnp.int32)
    dst_s, src_s, perm = lax.sort((dst, src, eid), num_keys=1)
    bounds = jnp.searchsorted(
        dst_s, (jnp.arange(NG + 1, dtype=jnp.int32) * GSZ).astype(jnp.int32)
    ).astype(jnp.int32)
    bounds = jnp.pad(bounds, (0, NB - (NG + 1)), constant_values=N_EDGES)
    bounds_f = bounds.astype(jnp.float32)
    zpad = jnp.zeros((2 * EBLK,), jnp.int32)
    dstw_p = jnp.concatenate([(dst_s * AW).astype(jnp.float32),
                              jnp.zeros((2 * EBLK,), jnp.float32)])
    src_sp = jnp.concatenate([src_s, zpad])
    perm_p = jnp.concatenate([perm, zpad])

    folds = [
        _fold_layer(W_enc[i], b_enc[i], W_pre[i], b_pre[i],
                    W_post[i], b_post[i], W_edge, b_edge)
        for i in range(LAYERS)
    ]

    Wd_list, bd_list = [], []
    for i in range(LAYERS):
        Cea, bDe = folds[i][2], folds[i][3]
        bpad = _padw(bDe, W)
        if i == 0:
            bpad = bpad.at[DEGC].set(1.0)
        Wd_list.append(_padw(Cea, W))
        bd_list.append(bpad)
    Wcat = jnp.stack(Wd_list)
    bcat = jnp.stack(bd_list)
    W1c = jnp.matmul(W_edge, W1[80:], precision=HIGH)
    b1c = jnp.matmul(b_edge[None], W1[80:], precision=HIGH)[0] + b1
    Wea = _padw(W1c, GW)
    bea = _padw(b1c, GW)

    D0, D1, EA1 = _pre_edge(edge_attr, Wcat, bcat, Wea, bea)
    h, Q = _pre_node(x, W_node, b_node, _padw(folds[0][1], W))

    deg = None
    for i in range(LAYERS):
        Dc = D0 if i == 0 else D1
        s1f, s2f, mnf, mxf = _sc_segment(Q, Dc, dstw_p, src_sp, perm_p,
                                         bounds_f)
        s1r = s1f.reshape(NNP, AW)
        s2r = s2f.reshape(NNP, AW)
        mnr = mnf.reshape(NNP, AW)
        mxr = mxf.reshape(NNP, AW)
        if deg is None:
            deg = s1r[:N_NODES, DEGC:DEGC + 1]
        S1 = s1r[:N_NODES, :200]
        S2 = s2r[:N_NODES, :200]
        MN = mnr[:N_NODES, :200]
        MX = mxr[:N_NODES, :200]

        Wd, Ws, _, _, Wpx, WA, WB, WC, bp = folds[i]
        c = _node_a(h, S1, S2, MN, MX, deg, Wd, Wpx, WA, WB, WC, bp,
                    W_lin[i], b_lin[i])
        if i + 1 < LAYERS:
            Wo0 = _padw(folds[i + 1][1], W)
            Wo1 = jnp.zeros((F, 8), jnp.float32)
            h, Q, _ = _node_b(h, c, bn_g[i], bn_b[i], Wo0, Wo1,
                              relu_out=False)
        else:
            Wo0 = _padw(W1[:40], GW)
            Wo1 = _padw(W1[40:80], GW)
            h, G1, G2 = _node_b(h, c, bn_g[i], bn_b[i], Wo0, Wo1,
                                relu_out=True)

    U12 = _sc_edge_gather(G1, G2, src, dst).reshape(N_EDGES, GW)
    W2p = jnp.pad(W2, ((0, GW - 50), (0, 0)))
    return _edge_final(U12, EA1, W2p, b2, W3, b3)


# speedup vs baseline: 4.3188x; 1.0182x over previous
import functools
import jax
import jax.numpy as jnp
import numpy as np
from jax import lax
from jax.experimental import pallas as pl
from jax.experimental.pallas import tpu as pltpu
from jax.experimental.pallas import tpu_sc as plsc

N_NODES = 10000
N_EDGES = 320000
TOWERS = 5
F = 40
LAYERS = 2
DEG_HIST = np.zeros(64, dtype=np.float64)
DEG_HIST[32] = float(N_NODES)
AVG_LOG = float((np.log(np.arange(64) + 1.0) * DEG_HIST).sum() / DEG_HIST.sum())

NG = 160
GSZ = 64
NNP = NG * GSZ
W = 256
AW = 208
DEGC = 200
EBLK = 64
EP = N_EDGES + EBLK
NB = 192
GW = 128
HIGH = jax.lax.Precision.HIGHEST


def _mm(a, b):
    return jnp.matmul(a, b, precision=HIGH)


_sc_mesh = plsc.VectorSubcoreMesh(core_axis_name="c", subcore_axis_name="s")


@functools.partial(
    pl.kernel,
    mesh=_sc_mesh,
    out_type=[jax.ShapeDtypeStruct((NNP * AW,), jnp.float32)] * 4,
    scratch_types=[
        pltpu.VMEM((GSZ * AW,), jnp.float32),
        pltpu.VMEM((GSZ * AW,), jnp.float32),
        pltpu.VMEM((GSZ * AW,), jnp.float32),
        pltpu.VMEM((GSZ * AW,), jnp.float32),
        pltpu.VMEM((EBLK,), jnp.int32),
        pltpu.VMEM((EBLK,), jnp.int32),
        pltpu.VMEM((EBLK, W), jnp.float32),
        pltpu.VMEM((EBLK, W), jnp.float32),
        pltpu.VMEM((EBLK + 16,), jnp.float32),
        pltpu.VMEM((EBLK,), jnp.int32),
        pltpu.VMEM((EBLK,), jnp.int32),
        pltpu.VMEM((EBLK, W), jnp.float32),
        pltpu.VMEM((EBLK, W), jnp.float32),
        pltpu.VMEM((EBLK + 16,), jnp.float32),
        pltpu.VMEM((NB,), jnp.float32),
        pltpu.SemaphoreType.DMA,
        pltpu.SemaphoreType.DMA,
        pltpu.SemaphoreType.DMA,
        pltpu.SemaphoreType.DMA,
    ],
)
def _sc_segment(q, d, dsts, srcs, perms, bounds,
                s1o, s2o, mno, mxo,
                a1, a2, a3, a4,
                srcvA, permvA, qbufA, dbufA, dstvA,
                srcvB, permvB, qbufB, dbufB, dstvB,
                bndv, semqA, semdA, semqB, semdB):
    wid = lax.axis_index("c") * 16 + lax.axis_index("s")
    pltpu.sync_copy(bounds, bndv)
    zero = jnp.zeros((16,), jnp.float32)
    big = jnp.full((16,), 1e30, jnp.float32)

    bufA = (srcvA, permvA, qbufA, dbufA, dstvA, semqA, semdA)
    bufB = (srcvB, permvB, qbufB, dbufB, dstvB, semqB, semdB)

    def _extract(vecref, idx):
        return vecref[pl.ds(idx, 16)][0]

    def _bval(idx):
        return _extract(bndv, idx).astype(jnp.int32)

    def _fire(buf, b):
        srcv, permv, qbuf, dbuf, dstv, semq, semd = buf
        eb = b * EBLK
        pltpu.sync_copy(dsts.at[pl.ds(eb, EBLK)], dstv.at[pl.ds(0, EBLK)])
        pltpu.sync_copy(srcs.at[pl.ds(eb, EBLK)], srcv)
        pltpu.sync_copy(perms.at[pl.ds(eb, EBLK)], permv)
        pltpu.async_copy(q.at[srcv], qbuf, semq)
        pltpu.async_copy(d.at[permv], dbuf, semd)

    def _wait(buf):
        srcv, permv, qbuf, dbuf, dstv, semq, semd = buf
        pltpu.make_async_copy(q.at[srcv], qbuf, semq).wait()
        pltpu.make_async_copy(d.at[permv], dbuf, semd).wait()

    for gi in range(NG // 32):
        g = wid * (NG // 32) + gi
        gbase = g * GSZ
        e0 = _bval(g)
        e1 = _bval(g + 1)

        def init_body(j, _):
            o = j * 16
            a1[pl.ds(o, 16)] = zero
            a2[pl.ds(o, 16)] = zero
            a3[pl.ds(o, 16)] = big
            a4[pl.ds(o, 16)] = -big
            return 0

        lax.fori_loop(0, GSZ * AW // 16, init_body, 0)

        b0 = e0 // EBLK
        b1 = (e1 + EBLK - 1) // EBLK

        def _process(buf, b):
            srcv, permv, qbuf, dbuf, dstv, semq, semd = buf
            eb = b * EBLK
            lo = jnp.maximum(e0 - eb, 0)
            hi = jnp.minimum(jnp.int32(EBLK), e1 - eb)

            def edge_body(i, _):
                off = _extract(dstv, i).astype(jnp.int32) - gbase * AW
                for c in range(AW // 16):
                    u = qbuf[i, pl.ds(c * 16, 16)] + dbuf[i, pl.ds(c * 16, 16)]
                    o = off + c * 16
                    plsc.addupdate(a1.at[pl.ds(o, 16)], u)
                    plsc.addupdate(a2.at[pl.ds(o, 16)], u * u)
                    a3[pl.ds(o, 16)] = jnp.minimum(a3[pl.ds(o, 16)], u)
                    a4[pl.ds(o, 16)] = jnp.maximum(a4[pl.ds(o, 16)], u)
                return 0

            lax.fori_loop(lo, hi, edge_body, 0)

        _fire(bufA, b0)

        def pair_body(t, _):
            b = b0 + 2 * t
            _fire(bufB, b + 1)
            _wait(bufA)
            _process(bufA, b)
            _fire(bufA, b + 2)
            _wait(bufB)
            _process(bufB, b + 1)
            return 0

        npairs = (b1 - b0 + 1) // 2
        lax.fori_loop(0, npairs, pair_body, 0)
        _wait(bufA)

        obase = gbase * AW
        pltpu.sync_copy(a1, s1o.at[pl.ds(obase, GSZ * AW)])
        pltpu.sync_copy(a2, s2o.at[pl.ds(obase, GSZ * AW)])
        pltpu.sync_copy(a3, mno.at[pl.ds(obase, GSZ * AW)])
        pltpu.sync_copy(a4, mxo.at[pl.ds(obase, GSZ * AW)])


GBLK = 128
NBLK_G = N_EDGES // GBLK
NPER_G = (NBLK_G + 31) // 32
GPAD = (NPER_G * 32 + 2 - NBLK_G) * GBLK


@functools.partial(
    pl.kernel,
    mesh=_sc_mesh,
    out_type=jax.ShapeDtypeStruct((N_EDGES * GW,), jnp.float32),
    scratch_types=[
        pltpu.VMEM((GBLK,), jnp.int32),
        pltpu.VMEM((GBLK,), jnp.int32),
        pltpu.VMEM((GBLK, GW), jnp.float32),
        pltpu.VMEM((GBLK, GW), jnp.float32),
        pltpu.VMEM((GBLK,), jnp.int32),
        pltpu.VMEM((GBLK,), jnp.int32),
        pltpu.VMEM((GBLK, GW), jnp.float32),
        pltpu.VMEM((GBLK, GW), jnp.float32),
        pltpu.VMEM((GBLK * GW,), jnp.float32),
        pltpu.SemaphoreType.DMA,
        pltpu.SemaphoreType.DMA,
        pltpu.SemaphoreType.DMA,
        pltpu.SemaphoreType.DMA,
    ],
)
def _sc_edge_gather(g1, g2, srcs, dsts, out,
                    srcvA, dstvA, b1A, b2A,
                    srcvB, dstvB, b1B, b2B,
                    wbuf, semsA, semdA, semsB, semdB):
    wid = lax.axis_index("c") * 16 + lax.axis_index("s")
    base = wid * NPER_G
    bufA = (srcvA, dstvA, b1A, b2A, semsA, semdA)
    bufB = (srcvB, dstvB, b1B, b2B, semsB, semdB)

    def _fire(buf, b):
        srcv, dstv, b1, b2, sems, semd = buf
        eb = b * GBLK
        pltpu.sync_copy(srcs.at[pl.ds(eb, GBLK)], srcv)
        pltpu.sync_copy(dsts.at[pl.ds(eb, GBLK)], dstv)
        pltpu.async_copy(g1.at[srcv], b1, sems)
        pltpu.async_copy(g2.at[dstv], b2, semd)

    def _wait(buf):
        srcv, dstv, b1, b2, sems, semd = buf
        pltpu.make_async_copy(g1.at[srcv], b1, sems).wait()
        pltpu.make_async_copy(g2.at[dstv], b2, semd).wait()

    def _process(buf, b):
        srcv, dstv, b1, b2, sems, semd = buf

        @pl.when(b < NBLK_G)
        def _():
            eb = b * GBLK

            def add_body(j, _):
                o = j * 16
                r = j // (GW // 16)
                c = (j % (GW // 16)) * 16
                wbuf[pl.ds(o, 16)] = b1[r, pl.ds(c, 16)] + b2[r, pl.ds(c, 16)]
                return 0

            lax.fori_loop(0, GBLK * GW // 16, add_body, 0)
            pltpu.sync_copy(wbuf, out.at[pl.ds(eb * GW, GBLK * GW)])

    _fire(bufA, base)

    def pair_body(t, _):
        b = base + 2 * t
        _fire(bufB, b + 1)
        _wait(bufA)
        _process(bufA, b)
        _fire(bufA, b + 2)
        _wait(bufB)
        _process(bufB, b + 1)
        return 0

    lax.fori_loop(0, (NPER_G + 1) // 2, pair_body, 0)
    _wait(bufA)


def _pre_node_body(x_ref, wn_ref, bn_ref, wq_ref, h_ref, q_ref):
    h = _mm(x_ref[...], wn_ref[...]) + bn_ref[...]
    h_ref[...] = h
    q_ref[...] = _mm(h, wq_ref[...])


def _pre_node(x, Wn, bn, Wq):
    return pl.pallas_call(
        _pre_node_body,
        out_shape=[
            jax.ShapeDtypeStruct((N_NODES, F), jnp.float32),
            jax.ShapeDtypeStruct((N_NODES, W), jnp.float32),
        ],
    )(x, Wn, bn, Wq)


def _pre_edge_body(ea_ref, w_ref, b_ref, wea_ref, bea_ref, d0, d1, ea1):
    a = ea_ref[...]
    w = w_ref[...]
    b = b_ref[...]
    d0[...] = _mm(a, w[0]) + b[0]
    d1[...] = _mm(a, w[1]) + b[1]
    ea1[...] = _mm(a, wea_ref[...]) + bea_ref[...]


def _pre_edge(edge_attr, Wcat, bcat, Wea, bea):
    BLK = 4000
    grid = (N_EDGES // BLK,)
    return pl.pallas_call(
        _pre_edge_body,
        grid=grid,
        in_specs=[
            pl.BlockSpec((BLK, 16), lambda i: (i, 0)),
            pl.BlockSpec(Wcat.shape, lambda i: (0, 0, 0)),
            pl.BlockSpec(bcat.shape, lambda i: (0, 0)),
            pl.BlockSpec(Wea.shape, lambda i: (0, 0)),
            pl.BlockSpec(bea.shape, lambda i: (0,)),
        ],
        out_specs=[pl.BlockSpec((BLK, W), lambda i: (i, 0))] * 2
        + [pl.BlockSpec((BLK, GW), lambda i: (i, 0))],
        out_shape=[jax.ShapeDtypeStruct((N_EDGES, W), jnp.float32)] * 2
        + [jax.ShapeDtypeStruct((N_EDGES, GW), jnp.float32)],
    )(edge_attr, Wcat, bcat, Wea, bea)


def _node_a_body(h_ref, s1_ref, s2_ref, mn_ref, mx_ref, deg_ref,
                 wd_ref, wpx_ref, wa_ref, wb_ref, wc_ref, bp_ref,
                 wl_ref, bl_ref, c_ref):
    h = h_ref[...]
    deg = deg_ref[...]
    has = deg > 0
    degc = jnp.maximum(deg, 1.0)
    P = _mm(h, wd_ref[...])
    S1 = s1_ref[...]
    S2 = s2_ref[...]
    mean = jnp.where(has, P + S1 / degc, 0.0)
    mn = jnp.where(has, P + mn_ref[...], 0.0)
    mx = jnp.where(has, P + mx_ref[...], 0.0)
    s1d = S1 / degc
    std = jnp.sqrt(jax.nn.relu(S2 / degc - s1d * s1d) + 1e-5)
    lg = jnp.log(degc + 1.0)
    amp = lg / AVG_LOG
    att = AVG_LOG / lg
    parts = []
    for t in range(TOWERS):
        sl = slice(t * F, (t + 1) * F)
        parts.extend([mean[:, sl], mn[:, sl], mx[:, sl], std[:, sl]])
    agg = jnp.concatenate(parts, axis=-1)
    c = (_mm(h, wpx_ref[...]) + _mm(agg, wa_ref[...])
         + amp * _mm(agg, wb_ref[...]) + att * _mm(agg, wc_ref[...]) + bp_ref[...])
    c_ref[...] = _mm(c, wl_ref[...]) + bl_ref[...]


def _node_a(h, S1, S2, MN, MX, deg, Wd, Wpx, WA, WB, WC, bp, Wl, bl):
    BLK = 1000
    grid = (N_NODES // BLK,)
    return pl.pallas_call(
        _node_a_body,
        grid=grid,
        in_specs=[
            pl.BlockSpec((BLK, F), lambda i: (i, 0)),
            pl.BlockSpec((BLK, 200), lambda i: (i, 0)),
            pl.BlockSpec((BLK, 200), lambda i: (i, 0)),
            pl.BlockSpec((BLK, 200), lambda i: (i, 0)),
            pl.BlockSpec((BLK, 200), lambda i: (i, 0)),
            pl.BlockSpec((BLK, 1), lambda i: (i, 0)),
            pl.BlockSpec(Wd.shape, lambda i: (0, 0)),
            pl.BlockSpec(Wpx.shape, lambda i: (0, 0)),
            pl.BlockSpec(WA.shape, lambda i: (0, 0)),
            pl.BlockSpec(WB.shape, lambda i: (0, 0)),
            pl.BlockSpec(WC.shape, lambda i: (0, 0)),
            pl.BlockSpec(bp.shape, lambda i: (0,)),
            pl.BlockSpec(Wl.shape, lambda i: (0, 0)),
            pl.BlockSpec(bl.shape, lambda i: (0,)),
        ],
        out_specs=pl.BlockSpec((BLK, F), lambda i: (i, 0)),
        out_shape=jax.ShapeDtypeStruct((N_NODES, F), jnp.float32),
    )(h, S1, S2, MN, MX, deg, Wd, Wpx, WA, WB, WC, bp, Wl, bl)


def _node_b_body(h_ref, c_ref, g_ref, b_ref, wo0_ref, wo1_ref,
                 hn_ref, o0_ref, o1_ref, *, relu_out):
    c = c_ref[...]
    mu = jnp.mean(c, axis=0, keepdims=True)
    var = jnp.mean((c - mu) ** 2, axis=0, keepdims=True)
    cbn = g_ref[...] * (c - mu) / jnp.sqrt(var + 1e-5) + b_ref[...]
    hn = (h_ref[...] + jax.nn.relu(cbn)) / 2.0
    hn_ref[...] = hn
    src_h = jax.nn.relu(hn) if relu_out else hn
    o0_ref[...] = _mm(src_h, wo0_ref[...])
    o1_ref[...] = _mm(src_h, wo1_ref[...])


def _node_b(h, c, bn_g, bn_b, Wo0, Wo1, relu_out):
    return pl.pallas_call(
        functools.partial(_node_b_body, relu_out=relu_out),
        out_shape=[
            jax.ShapeDtypeStruct((N_NODES, F), jnp.float32),
            jax.ShapeDtypeStruct((N_NODES, Wo0.shape[1]), jnp.float32),
            jax.ShapeDtypeStruct((N_NODES, Wo1.shape[1]), jnp.float32),
        ],
    )(h, c, bn_g, bn_b, Wo0, Wo1)


def _edge_final_body(u_ref, ea1_ref, w2_ref, b2_ref, w3_ref, b3_ref, out_ref):
    z = jax.nn.relu(u_ref[...] + ea1_ref[...])
    z = jax.nn.relu(_mm(z, w2_ref[...]) + b2_ref[...])
    out_ref[...] = _mm(z, w3_ref[...]) + b3_ref[...]


def _edge_final(U12, EA1, W2p, b2, W3, b3):
    BLK = 8000
    grid = (N_EDGES // BLK,)
    return pl.pallas_call(
        _edge_final_body,
        grid=grid,
        in_specs=[
            pl.BlockSpec((BLK, GW), lambda i: (i, 0)),
            pl.BlockSpec((BLK, GW), lambda i: (i, 0)),
            pl.BlockSpec(W2p.shape, lambda i: (0, 0)),
            pl.BlockSpec(b2.shape, lambda i: (0,)),
            pl.BlockSpec(W3.shape, lambda i: (0, 0)),
            pl.BlockSpec(b3.shape, lambda i: (0,)),
        ],
        out_specs=pl.BlockSpec((BLK, 2), lambda i: (i, 0)),
        out_shape=jax.ShapeDtypeStruct((N_EDGES, 2), jnp.float32),
    )(U12, EA1, W2p, b2, W3, b3)


def _fold_layer(W_enc, b_enc, W_pre, b_pre, W_post, b_post, W_edge, b_edge):
    Wd = jnp.concatenate([W_pre[t][:F] for t in range(TOWERS)], axis=1)
    Ws = jnp.concatenate([W_pre[t][F:2*F] for t in range(TOWERS)], axis=1)
    We = [W_pre[t][2*F:] for t in range(TOWERS)]
    C = jnp.concatenate([jnp.matmul(W_enc, We[t], precision=HIGH)
                         for t in range(TOWERS)], axis=1)
    bD = jnp.concatenate([jnp.matmul(b_enc[None], We[t], precision=HIGH)[0]
                          + b_pre[t] for t in range(TOWERS)])
    Cea = jnp.matmul(W_edge, C, precision=HIGH)
    bDe = jnp.matmul(b_edge[None], C, precision=HIGH)[0] + bD
    Wpx = jnp.concatenate([W_post[t][:F] for t in range(TOWERS)], axis=1)
    WA = jax.scipy.linalg.block_diag(*[W_post[t][F:F+4*F] for t in range(TOWERS)])
    WB = jax.scipy.linalg.block_diag(*[W_post[t][F+4*F:F+8*F] for t in range(TOWERS)])
    WC = jax.scipy.linalg.block_diag(*[W_post[t][F+8*F:] for t in range(TOWERS)])
    bp = jnp.concatenate([b_post[t] for t in range(TOWERS)])
    return Wd, Ws, Cea, bDe, Wpx, WA, WB, WC, bp


def _padw(M, width):
    pad = [(0, 0)] * (M.ndim - 1) + [(0, width - M.shape[-1])]
    return jnp.pad(M, pad)


def kernel(x, edge_attr, W_node, b_node, W_edge, b_edge, W_enc, b_enc, W_pre, b_pre, W_post, b_post, W_lin, b_lin, bn_g, bn_b, W1, b1, W2, b2, W3, b3, edge_index):
    src = edge_index[0]
    dst = edge_index[1]

    eid = jnp.arange(N_EDGES, dtype=jnp.int32)
    dst_s, src_s, perm = lax.sort((dst, src, eid), num_keys=1)
    bounds = jnp.searchsorted(
        dst_s, (jnp.arange(NG + 1, dtype=jnp.int32) * GSZ).astype(jnp.int32)
    ).astype(jnp.int32)
    bounds = jnp.pad(bounds, (0, NB - (NG + 1)), constant_values=N_EDGES)
    bounds_f = bounds.astype(jnp.float32)
    zpad = jnp.zeros((2 * EBLK,), jnp.int32)
    dstw_p = jnp.concatenate([(dst_s * AW).astype(jnp.float32),
                              jnp.zeros((2 * EBLK,), jnp.float32)])
    src_sp = jnp.concatenate([src_s, zpad])
    perm_p = jnp.concatenate([perm, zpad])

    folds = [
        _fold_layer(W_enc[i], b_enc[i], W_pre[i], b_pre[i],
                    W_post[i], b_post[i], W_edge, b_edge)
        for i in range(LAYERS)
    ]

    Wd_list, bd_list = [], []
    for i in range(LAYERS):
        Cea, bDe = folds[i][2], folds[i][3]
        bpad = _padw(bDe, W)
        if i == 0:
            bpad = bpad.at[DEGC].set(1.0)
        Wd_list.append(_padw(Cea, W))
        bd_list.append(bpad)
    Wcat = jnp.stack(Wd_list)
    bcat = jnp.stack(bd_list)
    W1c = jnp.matmul(W_edge, W1[80:], precision=HIGH)
    b1c = jnp.matmul(b_edge[None], W1[80:], precision=HIGH)[0] + b1
    Wea = _padw(W1c, GW)
    bea = _padw(b1c, GW)

    D0, D1, EA1 = _pre_edge(edge_attr, Wcat, bcat, Wea, bea)
    h, Q = _pre_node(x, W_node, b_node, _padw(folds[0][1], W))

    deg = None
    for i in range(LAYERS):
        Dc = D0 if i == 0 else D1
        s1f, s2f, mnf, mxf = _sc_segment(Q, Dc, dstw_p, src_sp, perm_p,
                                         bounds_f)
        s1r = s1f.reshape(NNP, AW)
        s2r = s2f.reshape(NNP, AW)
        mnr = mnf.reshape(NNP, AW)
        mxr = mxf.reshape(NNP, AW)
        if deg is None:
            deg = s1r[:N_NODES, DEGC:DEGC + 1]
        S1 = s1r[:N_NODES, :200]
        S2 = s2r[:N_NODES, :200]
        MN = mnr[:N_NODES, :200]
        MX = mxr[:N_NODES, :200]

        Wd, Ws, _, _, Wpx, WA, WB, WC, bp = folds[i]
        c = _node_a(h, S1, S2, MN, MX, deg, Wd, Wpx, WA, WB, WC, bp,
                    W_lin[i], b_lin[i])
        if i + 1 < LAYERS:
            Wo0 = _padw(folds[i + 1][1], W)
            Wo1 = jnp.zeros((F, 8), jnp.float32)
            h, Q, _ = _node_b(h, c, bn_g[i], bn_b[i], Wo0, Wo1,
                              relu_out=False)
        else:
            Wo0 = _padw(W1[:40], GW)
            Wo1 = _padw(W1[40:80], GW)
            h, G1, G2 = _node_b(h, c, bn_g[i], bn_b[i], Wo0, Wo1,
                                relu_out=True)

    gp = jnp.zeros((GPAD,), jnp.int32)
    U12 = _sc_edge_gather(G1, G2, jnp.concatenate([src, gp]),
                          jnp.concatenate([dst, gp])).reshape(N_EDGES, GW)
    W2p = jnp.pad(W2, ((0, GW - 50), (0, 0)))
    return _edge_final(U12, EA1, W2p, b2, W3, b3)


# speedup vs baseline: 4.9559x; 1.1475x over previous
import functools
import jax
import jax.numpy as jnp
import numpy as np
from jax import lax
from jax.experimental import pallas as pl
from jax.experimental.pallas import tpu as pltpu
from jax.experimental.pallas import tpu_sc as plsc

N_NODES = 10000
N_EDGES = 320000
TOWERS = 5
F = 40
LAYERS = 2
DEG_HIST = np.zeros(64, dtype=np.float64)
DEG_HIST[32] = float(N_NODES)
AVG_LOG = float((np.log(np.arange(64) + 1.0) * DEG_HIST).sum() / DEG_HIST.sum())

NG = 160
GSZ = 64
NNP = NG * GSZ
W = 256
AW = 208
DEGC = 200
EBLK = 64
EP = N_EDGES + EBLK
NB = 192
GW = 128
HIGH = jax.lax.Precision.HIGHEST


def _mm(a, b):
    return jnp.matmul(a, b)


_sc_mesh = plsc.VectorSubcoreMesh(core_axis_name="c", subcore_axis_name="s")


@functools.partial(
    pl.kernel,
    mesh=_sc_mesh,
    out_type=[jax.ShapeDtypeStruct((NNP * AW,), jnp.float32)] * 4,
    scratch_types=[
        pltpu.VMEM((GSZ * AW,), jnp.float32),
        pltpu.VMEM((GSZ * AW,), jnp.float32),
        pltpu.VMEM((GSZ * AW,), jnp.float32),
        pltpu.VMEM((GSZ * AW,), jnp.float32),
        pltpu.VMEM((EBLK,), jnp.int32),
        pltpu.VMEM((EBLK,), jnp.int32),
        pltpu.VMEM((EBLK, W), jnp.float32),
        pltpu.VMEM((EBLK, W), jnp.float32),
        pltpu.VMEM((EBLK + 16,), jnp.float32),
        pltpu.VMEM((EBLK,), jnp.int32),
        pltpu.VMEM((EBLK,), jnp.int32),
        pltpu.VMEM((EBLK, W), jnp.float32),
        pltpu.VMEM((EBLK, W), jnp.float32),
        pltpu.VMEM((EBLK + 16,), jnp.float32),
        pltpu.VMEM((NB,), jnp.float32),
        pltpu.SemaphoreType.DMA,
        pltpu.SemaphoreType.DMA,
        pltpu.SemaphoreType.DMA,
        pltpu.SemaphoreType.DMA,
    ],
)
def _sc_segment(q, d, dsts, srcs, perms, bounds,
                s1o, s2o, mno, mxo,
                a1, a2, a3, a4,
                srcvA, permvA, qbufA, dbufA, dstvA,
                srcvB, permvB, qbufB, dbufB, dstvB,
                bndv, semqA, semdA, semqB, semdB):
    wid = lax.axis_index("c") * 16 + lax.axis_index("s")
    pltpu.sync_copy(bounds, bndv)
    zero = jnp.zeros((16,), jnp.float32)
    big = jnp.full((16,), 1e30, jnp.float32)

    bufA = (srcvA, permvA, qbufA, dbufA, dstvA, semqA, semdA)
    bufB = (srcvB, permvB, qbufB, dbufB, dstvB, semqB, semdB)

    def _extract(vecref, idx):
        return vecref[pl.ds(idx, 16)][0]

    def _bval(idx):
        return _extract(bndv, idx).astype(jnp.int32)

    def _fire(buf, b):
        srcv, permv, qbuf, dbuf, dstv, semq, semd = buf
        eb = b * EBLK
        pltpu.sync_copy(dsts.at[pl.ds(eb, EBLK)], dstv.at[pl.ds(0, EBLK)])
        pltpu.sync_copy(srcs.at[pl.ds(eb, EBLK)], srcv)
        pltpu.sync_copy(perms.at[pl.ds(eb, EBLK)], permv)
        pltpu.async_copy(q.at[srcv], qbuf, semq)
        pltpu.async_copy(d.at[permv], dbuf, semd)

    def _wait(buf):
        srcv, permv, qbuf, dbuf, dstv, semq, semd = buf
        pltpu.make_async_copy(q.at[srcv], qbuf, semq).wait()
        pltpu.make_async_copy(d.at[permv], dbuf, semd).wait()

    for gi in range(NG // 32):
        g = wid * (NG // 32) + gi
        gbase = g * GSZ
        e0 = _bval(g)
        e1 = _bval(g + 1)

        def init_body(j, _):
            o = j * 16
            a1[pl.ds(o, 16)] = zero
            a2[pl.ds(o, 16)] = zero
            a3[pl.ds(o, 16)] = big
            a4[pl.ds(o, 16)] = -big
            return 0

        lax.fori_loop(0, GSZ * AW // 16, init_body, 0)

        b0 = e0 // EBLK
        b1 = (e1 + EBLK - 1) // EBLK

        def _process(buf, b):
            srcv, permv, qbuf, dbuf, dstv, semq, semd = buf
            eb = b * EBLK
            lo = jnp.maximum(e0 - eb, 0)
            hi = jnp.minimum(jnp.int32(EBLK), e1 - eb)

            def edge_body(i, _):
                off = _extract(dstv, i).astype(jnp.int32) - gbase * AW
                for c in range(AW // 16):
                    u = qbuf[i, pl.ds(c * 16, 16)] + dbuf[i, pl.ds(c * 16, 16)]
                    o = off + c * 16
                    plsc.addupdate(a1.at[pl.ds(o, 16)], u)
                    plsc.addupdate(a2.at[pl.ds(o, 16)], u * u)
                    a3[pl.ds(o, 16)] = jnp.minimum(a3[pl.ds(o, 16)], u)
                    a4[pl.ds(o, 16)] = jnp.maximum(a4[pl.ds(o, 16)], u)
                return 0

            lax.fori_loop(lo, hi, edge_body, 0)

        _fire(bufA, b0)

        def pair_body(t, _):
            b = b0 + 2 * t
            _fire(bufB, b + 1)
            _wait(bufA)
            _process(bufA, b)
            _fire(bufA, b + 2)
            _wait(bufB)
            _process(bufB, b + 1)
            return 0

        npairs = (b1 - b0 + 1) // 2
        lax.fori_loop(0, npairs, pair_body, 0)
        _wait(bufA)

        obase = gbase * AW
        pltpu.sync_copy(a1, s1o.at[pl.ds(obase, GSZ * AW)])
        pltpu.sync_copy(a2, s2o.at[pl.ds(obase, GSZ * AW)])
        pltpu.sync_copy(a3, mno.at[pl.ds(obase, GSZ * AW)])
        pltpu.sync_copy(a4, mxo.at[pl.ds(obase, GSZ * AW)])


GBLK = 128
NBLK_G = N_EDGES // GBLK
NPER_G = (NBLK_G + 31) // 32
GPAD = (NPER_G * 32 + 2 - NBLK_G) * GBLK


@functools.partial(
    pl.kernel,
    mesh=_sc_mesh,
    out_type=jax.ShapeDtypeStruct((N_EDGES * GW,), jnp.float32),
    scratch_types=[
        pltpu.VMEM((GBLK,), jnp.int32),
        pltpu.VMEM((GBLK,), jnp.int32),
        pltpu.VMEM((GBLK, GW), jnp.float32),
        pltpu.VMEM((GBLK, GW), jnp.float32),
        pltpu.VMEM((GBLK,), jnp.int32),
        pltpu.VMEM((GBLK,), jnp.int32),
        pltpu.VMEM((GBLK, GW), jnp.float32),
        pltpu.VMEM((GBLK, GW), jnp.float32),
        pltpu.VMEM((GBLK * GW,), jnp.float32),
        pltpu.SemaphoreType.DMA,
        pltpu.SemaphoreType.DMA,
        pltpu.SemaphoreType.DMA,
        pltpu.SemaphoreType.DMA,
    ],
)
def _sc_edge_gather(g1, g2, srcs, dsts, out,
                    srcvA, dstvA, b1A, b2A,
                    srcvB, dstvB, b1B, b2B,
                    wbuf, semsA, semdA, semsB, semdB):
    wid = lax.axis_index("c") * 16 + lax.axis_index("s")
    base = wid * NPER_G
    bufA = (srcvA, dstvA, b1A, b2A, semsA, semdA)
    bufB = (srcvB, dstvB, b1B, b2B, semsB, semdB)

    def _fire(buf, b):
        srcv, dstv, b1, b2, sems, semd = buf
        eb = b * GBLK
        pltpu.sync_copy(srcs.at[pl.ds(eb, GBLK)], srcv)
        pltpu.sync_copy(dsts.at[pl.ds(eb, GBLK)], dstv)
        pltpu.async_copy(g1.at[srcv], b1, sems)
        pltpu.async_copy(g2.at[dstv], b2, semd)

    def _wait(buf):
        srcv, dstv, b1, b2, sems, semd = buf
        pltpu.make_async_copy(g1.at[srcv], b1, sems).wait()
        pltpu.make_async_copy(g2.at[dstv], b2, semd).wait()

    def _process(buf, b):
        srcv, dstv, b1, b2, sems, semd = buf

        @pl.when(b < NBLK_G)
        def _():
            eb = b * GBLK

            def add_body(j, _):
                o = j * 16
                r = j // (GW // 16)
                c = (j % (GW // 16)) * 16
                wbuf[pl.ds(o, 16)] = b1[r, pl.ds(c, 16)] + b2[r, pl.ds(c, 16)]
                return 0

            lax.fori_loop(0, GBLK * GW // 16, add_body, 0)
            pltpu.sync_copy(wbuf, out.at[pl.ds(eb * GW, GBLK * GW)])

    _fire(bufA, base)

    def pair_body(t, _):
        b = base + 2 * t
        _fire(bufB, b + 1)
        _wait(bufA)
        _process(bufA, b)
        _fire(bufA, b + 2)
        _wait(bufB)
        _process(bufB, b + 1)
        return 0

    lax.fori_loop(0, (NPER_G + 1) // 2, pair_body, 0)
    _wait(bufA)


def _pre_node_body(x_ref, wn_ref, bn_ref, wq_ref, h_ref, q_ref):
    h = _mm(x_ref[...], wn_ref[...]) + bn_ref[...]
    h_ref[...] = h
    q_ref[...] = _mm(h, wq_ref[...])


def _pre_node(x, Wn, bn, Wq):
    return pl.pallas_call(
        _pre_node_body,
        out_shape=[
            jax.ShapeDtypeStruct((N_NODES, F), jnp.float32),
            jax.ShapeDtypeStruct((N_NODES, W), jnp.float32),
        ],
    )(x, Wn, bn, Wq)


def _pre_edge_body(attr_ref, wedge_ref, bedge_ref, wenc_ref, benc_ref,
                   wec_ref, bprec_ref, wea_ref, bea_ref, d0, d1, ea1):
    ea = _mm(attr_ref[...], wedge_ref[...]) + bedge_ref[...]
    wenc = wenc_ref[...]
    benc = benc_ref[...]
    wec = wec_ref[...]
    bprec = bprec_ref[...]
    e0 = _mm(ea, wenc[0]) + benc[0]
    e1 = _mm(ea, wenc[1]) + benc[1]
    d0[...] = _mm(e0, wec[0]) + bprec[0]
    d1[...] = _mm(e1, wec[1]) + bprec[1]
    ea1[...] = _mm(ea, wea_ref[...]) + bea_ref[...]


def _pre_edge(edge_attr, W_edge, b_edge, W_enc, b_enc, WeC, bpreC, Wea, bea):
    BLK = 4000
    grid = (N_EDGES // BLK,)
    return pl.pallas_call(
        _pre_edge_body,
        grid=grid,
        in_specs=[
            pl.BlockSpec((BLK, 16), lambda i: (i, 0)),
            pl.BlockSpec(W_edge.shape, lambda i: (0, 0)),
            pl.BlockSpec(b_edge.shape, lambda i: (0,)),
            pl.BlockSpec(W_enc.shape, lambda i: (0, 0, 0)),
            pl.BlockSpec(b_enc.shape, lambda i: (0, 0)),
            pl.BlockSpec(WeC.shape, lambda i: (0, 0, 0)),
            pl.BlockSpec(bpreC.shape, lambda i: (0, 0)),
            pl.BlockSpec(Wea.shape, lambda i: (0, 0)),
            pl.BlockSpec(bea.shape, lambda i: (0,)),
        ],
        out_specs=[pl.BlockSpec((BLK, W), lambda i: (i, 0))] * 2
        + [pl.BlockSpec((BLK, GW), lambda i: (i, 0))],
        out_shape=[jax.ShapeDtypeStruct((N_EDGES, W), jnp.float32)] * 2
        + [jax.ShapeDtypeStruct((N_EDGES, GW), jnp.float32)],
    )(edge_attr, W_edge, b_edge, W_enc, b_enc, WeC, bpreC, Wea, bea)


def _node_a_body(h_ref, s1_ref, s2_ref, mn_ref, mx_ref, deg_ref,
                 wd_ref, wpx_ref, wa_ref, wb_ref, wc_ref, bp_ref,
                 wl_ref, bl_ref, c_ref):
    h = h_ref[...]
    deg = deg_ref[...]
    has = deg > 0
    degc = jnp.maximum(deg, 1.0)
    P = _mm(h, wd_ref[...])
    S1 = s1_ref[...]
    S2 = s2_ref[...]
    mean = jnp.where(has, P + S1 / degc, 0.0)
    mn = jnp.where(has, P + mn_ref[...], 0.0)
    mx = jnp.where(has, P + mx_ref[...], 0.0)
    s1d = S1 / degc
    std = jnp.sqrt(jax.nn.relu(S2 / degc - s1d * s1d) + 1e-5)
    lg = jnp.log(degc + 1.0)
    amp = lg / AVG_LOG
    att = AVG_LOG / lg
    parts = []
    for t in range(TOWERS):
        sl = slice(t * F, (t + 1) * F)
        parts.extend([mean[:, sl], mn[:, sl], mx[:, sl], std[:, sl]])
    agg = jnp.concatenate(parts, axis=-1)
    c = (_mm(h, wpx_ref[...]) + _mm(agg, wa_ref[...])
         + amp * _mm(agg, wb_ref[...]) + att * _mm(agg, wc_ref[...]) + bp_ref[...])
    c_ref[...] = _mm(c, wl_ref[...]) + bl_ref[...]


def _node_a(h, S1, S2, MN, MX, deg, Wd, Wpx, WA, WB, WC, bp, Wl, bl):
    BLK = 1000
    grid = (N_NODES // BLK,)
    return pl.pallas_call(
        _node_a_body,
        grid=grid,
        in_specs=[
            pl.BlockSpec((BLK, F), lambda i: (i, 0)),
            pl.BlockSpec((BLK, 200), lambda i: (i, 0)),
            pl.BlockSpec((BLK, 200), lambda i: (i, 0)),
            pl.BlockSpec((BLK, 200), lambda i: (i, 0)),
            pl.BlockSpec((BLK, 200), lambda i: (i, 0)),
            pl.BlockSpec((BLK, 1), lambda i: (i, 0)),
            pl.BlockSpec(Wd.shape, lambda i: (0, 0)),
            pl.BlockSpec(Wpx.shape, lambda i: (0, 0)),
            pl.BlockSpec(WA.shape, lambda i: (0, 0)),
            pl.BlockSpec(WB.shape, lambda i: (0, 0)),
            pl.BlockSpec(WC.shape, lambda i: (0, 0)),
            pl.BlockSpec(bp.shape, lambda i: (0,)),
            pl.BlockSpec(Wl.shape, lambda i: (0, 0)),
            pl.BlockSpec(bl.shape, lambda i: (0,)),
        ],
        out_specs=pl.BlockSpec((BLK, F), lambda i: (i, 0)),
        out_shape=jax.ShapeDtypeStruct((N_NODES, F), jnp.float32),
    )(h, S1, S2, MN, MX, deg, Wd, Wpx, WA, WB, WC, bp, Wl, bl)


def _node_b_body(h_ref, c_ref, g_ref, b_ref, wo0_ref, wo1_ref,
                 hn_ref, o0_ref, o1_ref, *, relu_out):
    c = c_ref[...]
    mu = jnp.mean(c, axis=0, keepdims=True)
    var = jnp.mean((c - mu) ** 2, axis=0, keepdims=True)
    cbn = g_ref[...] * (c - mu) / jnp.sqrt(var + 1e-5) + b_ref[...]
    hn = (h_ref[...] + jax.nn.relu(cbn)) / 2.0
    hn_ref[...] = hn
    src_h = jax.nn.relu(hn) if relu_out else hn
    o0_ref[...] = _mm(src_h, wo0_ref[...])
    o1_ref[...] = _mm(src_h, wo1_ref[...])


def _node_b(h, c, bn_g, bn_b, Wo0, Wo1, relu_out):
    return pl.pallas_call(
        functools.partial(_node_b_body, relu_out=relu_out),
        out_shape=[
            jax.ShapeDtypeStruct((N_NODES, F), jnp.float32),
            jax.ShapeDtypeStruct((N_NODES, Wo0.shape[1]), jnp.float32),
            jax.ShapeDtypeStruct((N_NODES, Wo1.shape[1]), jnp.float32),
        ],
    )(h, c, bn_g, bn_b, Wo0, Wo1)


def _edge_final_body(u_ref, ea1_ref, w2_ref, b2_ref, w3_ref, b3_ref, out_ref):
    z = jax.nn.relu(u_ref[...] + ea1_ref[...])
    z = jax.nn.relu(_mm(z, w2_ref[...]) + b2_ref[...])
    out_ref[...] = _mm(z, w3_ref[...]) + b3_ref[...]


def _edge_final(U12, EA1, W2p, b2, W3, b3):
    BLK = 8000
    grid = (N_EDGES // BLK,)
    return pl.pallas_call(
        _edge_final_body,
        grid=grid,
        in_specs=[
            pl.BlockSpec((BLK, GW), lambda i: (i, 0)),
            pl.BlockSpec((BLK, GW), lambda i: (i, 0)),
            pl.BlockSpec(W2p.shape, lambda i: (0, 0)),
            pl.BlockSpec(b2.shape, lambda i: (0,)),
            pl.BlockSpec(W3.shape, lambda i: (0, 0)),
            pl.BlockSpec(b3.shape, lambda i: (0,)),
        ],
        out_specs=pl.BlockSpec((BLK, 2), lambda i: (i, 0)),
        out_shape=jax.ShapeDtypeStruct((N_EDGES, 2), jnp.float32),
    )(U12, EA1, W2p, b2, W3, b3)


def _fold_layer(W_enc, b_enc, W_pre, b_pre, W_post, b_post, W_edge, b_edge):
    Wd = jnp.concatenate([W_pre[t][:F] for t in range(TOWERS)], axis=1)
    Ws = jnp.concatenate([W_pre[t][F:2*F] for t in range(TOWERS)], axis=1)
    WeC = jnp.concatenate([W_pre[t][2*F:] for t in range(TOWERS)], axis=1)
    bpreC = jnp.concatenate([b_pre[t] for t in range(TOWERS)])
    Wpx = jnp.concatenate([W_post[t][:F] for t in range(TOWERS)], axis=1)
    WA = jax.scipy.linalg.block_diag(*[W_post[t][F:F+4*F] for t in range(TOWERS)])
    WB = jax.scipy.linalg.block_diag(*[W_post[t][F+4*F:F+8*F] for t in range(TOWERS)])
    WC = jax.scipy.linalg.block_diag(*[W_post[t][F+8*F:] for t in range(TOWERS)])
    bp = jnp.concatenate([b_post[t] for t in range(TOWERS)])
    return Wd, Ws, WeC, bpreC, Wpx, WA, WB, WC, bp


def _padw(M, width):
    pad = [(0, 0)] * (M.ndim - 1) + [(0, width - M.shape[-1])]
    return jnp.pad(M, pad)


def kernel(x, edge_attr, W_node, b_node, W_edge, b_edge, W_enc, b_enc, W_pre, b_pre, W_post, b_post, W_lin, b_lin, bn_g, bn_b, W1, b1, W2, b2, W3, b3, edge_index):
    src = edge_index[0]
    dst = edge_index[1]

    eid = jnp.arange(N_EDGES, dtype=jnp.int32)
    dst_s, src_s, perm = lax.sort((dst, src, eid), num_keys=1)
    bounds = jnp.searchsorted(
        dst_s, (jnp.arange(NG + 1, dtype=jnp.int32) * GSZ).astype(jnp.int32)
    ).astype(jnp.int32)
    bounds = jnp.pad(bounds, (0, NB - (NG + 1)), constant_values=N_EDGES)
    bounds_f = bounds.astype(jnp.float32)
    zpad = jnp.zeros((2 * EBLK,), jnp.int32)
    dstw_p = jnp.concatenate([(dst_s * AW).astype(jnp.float32),
                              jnp.zeros((2 * EBLK,), jnp.float32)])
    src_sp = jnp.concatenate([src_s, zpad])
    perm_p = jnp.concatenate([perm, zpad])

    folds = [
        _fold_layer(W_enc[i], b_enc[i], W_pre[i], b_pre[i],
                    W_post[i], b_post[i], W_edge, b_edge)
        for i in range(LAYERS)
    ]

    WeC_pad = jnp.stack([_padw(folds[i][2], W) for i in range(LAYERS)])
    bpre_pad = jnp.stack(
        [_padw(folds[i][3], W).at[DEGC].set(1.0) if i == 0
         else _padw(folds[i][3], W) for i in range(LAYERS)])
    Wea = _padw(W1[80:], GW)
    bea = _padw(b1, GW)

    D0, D1, EA1 = _pre_edge(edge_attr, W_edge, b_edge, W_enc, b_enc,
                            WeC_pad, bpre_pad, Wea, bea)
    h, Q = _pre_node(x, W_node, b_node, _padw(folds[0][1], W))

    deg = None
    for i in range(LAYERS):
        Dc = D0 if i == 0 else D1
        s1f, s2f, mnf, mxf = _sc_segment(Q, Dc, dstw_p, src_sp, perm_p,
                                         bounds_f)
        s1r = s1f.reshape(NNP, AW)
        s2r = s2f.reshape(NNP, AW)
        mnr = mnf.reshape(NNP, AW)
        mxr = mxf.reshape(NNP, AW)
        if deg is None:
            deg = s1r[:N_NODES, DEGC:DEGC + 1]
        S1 = s1r[:N_NODES, :200]
        S2 = s2r[:N_NODES, :200]
        MN = mnr[:N_NODES, :200]
        MX = mxr[:N_NODES, :200]

        Wd, Ws, _, _, Wpx, WA, WB, WC, bp = folds[i]
        c = _node_a(h, S1, S2, MN, MX, deg, Wd, Wpx, WA, WB, WC, bp,
                    W_lin[i], b_lin[i])
        if i + 1 < LAYERS:
            Wo0 = _padw(folds[i + 1][1], W)
            Wo1 = jnp.zeros((F, 8), jnp.float32)
            h, Q, _ = _node_b(h, c, bn_g[i], bn_b[i], Wo0, Wo1,
                              relu_out=False)
        else:
            Wo0 = _padw(W1[:40], GW)
            Wo1 = _padw(W1[40:80], GW)
            h, G1, G2 = _node_b(h, c, bn_g[i], bn_b[i], Wo0, Wo1,
                                relu_out=True)

    gp = jnp.zeros((GPAD,), jnp.int32)
    U12 = _sc_edge_gather(G1, G2, jnp.concatenate([src, gp]),
                          jnp.concatenate([dst, gp])).reshape(N_EDGES, GW)
    W2p = jnp.pad(W2, ((0, GW - 50), (0, 0)))
    return _edge_final(U12, EA1, W2p, b2, W3, b3)
